# Initial kernel scaffold; baseline (speedup 1.0000x reference)
import jax
import jax.numpy as jnp
from jax.experimental import pallas as pl


def kernel(drug_feature, drug_adj, ibatch, gexpr_data, W1, b1, g1, be1, W2, b2, g2, be2, Wc1, bc1, gc1, bec1, Wc2, bc2):
    raise NotImplementedError("write your pallas kernel here")


# speedup vs baseline: 18.4577x; 18.4577x over previous
import functools

import jax
import jax.numpy as jnp
from jax import lax
from jax.experimental import pallas as pl
from jax.experimental.pallas import tpu as pltpu
from jax.experimental.pallas import tpu_sc as plsc

_N = 10000
_E = 320000
_D = 128
_B = 128
_NC = 2
_NS = 16
_NW = _NC * _NS
_EPT = _E // _NW
_WIN = 80
_NWIN = _EPT // _WIN
_RPS = _N // _NS
_ZCH = 624
_SEG = 320
_BM = 1000


def _sc_mesh():
    return plsc.VectorSubcoreMesh(
        core_axis_name="c", subcore_axis_name="s",
        num_cores=_NC, num_subcores=_NS)


def _deg_call(dst3):
    @functools.partial(
        pl.kernel,
        out_type=jax.ShapeDtypeStruct((_NC * _N,), jnp.float32),
        mesh=_sc_mesh(),
        scratch_types=[
            pltpu.VMEM((_NWIN, _WIN), jnp.int32),
            pltpu.VMEM((_WIN,), jnp.float32),
            pltpu.VMEM((_ZCH,), jnp.float32),
            pltpu.VMEM_SHARED((_N,), jnp.float32),
            pltpu.SemaphoreType.DMA,
        ],
    )
    def deg_kernel(dst_hbm, out_hbm, idx_v, ones_v, z_v, deg_sh, sem):
        c = lax.axis_index("c")
        s = lax.axis_index("s")
        w = s * _NC + c

        def fill_ones(i, _):
            ones_v[pl.ds(i * 16, 16)] = jnp.full((16,), 1.0, jnp.float32)
            return 0
        lax.fori_loop(0, _WIN // 16, fill_ones, 0)

        def fill_zeros(i, _):
            z_v[pl.ds(i * 16, 16)] = jnp.zeros((16,), jnp.float32)
            return 0
        lax.fori_loop(0, _ZCH // 16, fill_zeros, 0)

        pltpu.sync_copy(z_v, deg_sh.at[pl.ds(s * _ZCH, _ZCH)])

        @pl.when(s == 0)
        def _():
            pltpu.sync_copy(z_v.at[pl.ds(0, 16)],
                            deg_sh.at[pl.ds(_NS * _ZCH, _N - _NS * _ZCH)])
        plsc.subcore_barrier()

        pltpu.sync_copy(dst_hbm.at[w], idx_v)

        def win(i, _):
            pltpu.sync_copy(ones_v, deg_sh.at[idx_v.at[i]], add=True)
            return 0
        lax.fori_loop(0, _NWIN, win, 0)
        plsc.subcore_barrier()

        pltpu.sync_copy(deg_sh.at[pl.ds(s * _ZCH, _ZCH)], z_v)
        pltpu.sync_copy(z_v, out_hbm.at[pl.ds(c * _N + s * _ZCH, _ZCH)])

        @pl.when(s == 0)
        def _():
            tail = _N - _NS * _ZCH
            pltpu.sync_copy(deg_sh.at[pl.ds(_NS * _ZCH, tail)],
                            z_v.at[pl.ds(0, tail)])
            pltpu.sync_copy(z_v.at[pl.ds(0, tail)],
                            out_hbm.at[pl.ds(c * _N + _NS * _ZCH, tail)])

    return deg_kernel(dst3)


def _scatter_call(src3, dst3, y):
    @functools.partial(
        pl.kernel,
        out_type=jax.ShapeDtypeStruct((_NC, _N, _D), jnp.float32),
        mesh=_sc_mesh(),
        scratch_types=[
            pltpu.VMEM((_NWIN, _WIN), jnp.int32),
            pltpu.VMEM((_NWIN, _WIN), jnp.int32),
            pltpu.VMEM((_WIN, _D), jnp.float32),
            pltpu.VMEM_SHARED((_N, _D), jnp.float32),
            pltpu.SemaphoreType.DMA,
        ],
    )
    def scat_kernel(src_hbm, dst_hbm, y_hbm, out_hbm,
                    sidx_v, didx_v, rows_v, acc_sh, sem):
        c = lax.axis_index("c")
        s = lax.axis_index("s")
        w = s * _NC + c
        base = s * _ZCH
        chunks = [(base + k * _WIN, _WIN) for k in range(_ZCH // _WIN)]
        rem = _ZCH - (_ZCH // _WIN) * _WIN
        if rem:
            chunks.append((base + (_ZCH // _WIN) * _WIN, rem))

        def zrow(i, _):
            for j in range(_D // 16):
                rows_v[i, pl.ds(j * 16, 16)] = jnp.zeros((16,), jnp.float32)
            return 0
        lax.fori_loop(0, _WIN, zrow, 0)
        for off, ln in chunks:
            pltpu.sync_copy(rows_v.at[pl.ds(0, ln)],
                            acc_sh.at[pl.ds(off, ln)])

        tail_off = _NS * _ZCH
        tail = _N - tail_off

        @pl.when(s == 0)
        def _():
            pltpu.sync_copy(rows_v.at[pl.ds(0, tail)],
                            acc_sh.at[pl.ds(tail_off, tail)])
        plsc.subcore_barrier()

        pltpu.sync_copy(src_hbm.at[w], sidx_v)
        pltpu.sync_copy(dst_hbm.at[w], didx_v)

        def win(i, _):
            pltpu.async_copy(y_hbm.at[sidx_v.at[i]], rows_v, sem).wait()
            pltpu.sync_copy(rows_v, acc_sh.at[didx_v.at[i]], add=True)
            return 0
        lax.fori_loop(0, _NWIN, win, 0)
        plsc.subcore_barrier()

        for off, ln in chunks:
            pltpu.sync_copy(acc_sh.at[pl.ds(off, ln)],
                            rows_v.at[pl.ds(0, ln)])
            pltpu.sync_copy(rows_v.at[pl.ds(0, ln)],
                            out_hbm.at[c, pl.ds(off, ln)])

        @pl.when(s == 0)
        def _():
            pltpu.sync_copy(acc_sh.at[pl.ds(tail_off, tail)],
                            rows_v.at[pl.ds(0, tail)])
            pltpu.sync_copy(rows_v.at[pl.ds(0, tail)],
                            out_hbm.at[c, pl.ds(tail_off, tail)])

    return scat_kernel(src3, dst3, y)


def _segmax_call(r2, ibatch, a2, c2):
    @functools.partial(
        pl.kernel,
        out_type=jax.ShapeDtypeStruct((_NW, _B, _D), jnp.float32),
        mesh=_sc_mesh(),
        scratch_types=[
            pltpu.VMEM((_SEG, _D), jnp.float32),
            pltpu.VMEM((_SEG,), jnp.int32),
            pltpu.VMEM((_B, _D), jnp.float32),
            pltpu.VMEM((_D,), jnp.float32),
            pltpu.VMEM((_D,), jnp.float32),
            pltpu.SemaphoreType.DMA,
        ],
    )
    def seg_kernel(r_hbm, ib_hbm, a_hbm, c_hbm, out_hbm,
                   rows_v, seg_v, out_v, a_v, c_v, sem):
        c = lax.axis_index("c")
        s = lax.axis_index("s")
        w = s * _NC + c
        start = jnp.minimum(w * _SEG, _N - _SEG)

        pltpu.sync_copy(a_hbm, a_v)
        pltpu.sync_copy(c_hbm, c_v)
        pltpu.sync_copy(r_hbm.at[pl.ds(start, _SEG)], rows_v)
        pltpu.sync_copy(ib_hbm.at[pl.ds(start, _SEG)], seg_v)

        neg = jnp.full((16,), -jnp.inf, jnp.float32)

        def initrow(i, _):
            for j in range(_D // 16):
                out_v[i, pl.ds(j * 16, 16)] = neg
            return 0
        lax.fori_loop(0, _B, initrow, 0)

        def row16(k, _):
            base16 = k * 16
            seg16 = seg_v[pl.ds(base16, 16)]
            for r in range(16):
                seg = seg16[r]
                for j in range(_D // 16):
                    sl = pl.ds(j * 16, 16)
                    val = rows_v[base16 + r, sl] * a_v[sl] + c_v[sl]
                    out_v[seg, sl] = jnp.maximum(out_v[seg, sl], val)
            return 0
        lax.fori_loop(0, _SEG // 16, row16, 0)

        pltpu.sync_copy(out_v, out_hbm.at[w])

    return seg_kernel(r2, ibatch, a2, c2)


def _mm_scaled(x, W, dinv_col, a_row, c_row):
    def kern(x_ref, w_ref, d_ref, a_ref, c_ref, o_ref):
        xb = x_ref[...] * a_ref[...] + c_ref[...]
        o_ref[...] = d_ref[...] * jnp.dot(
            xb, w_ref[...], preferred_element_type=jnp.float32)

    return pl.pallas_call(
        kern,
        grid=(_N // _BM,),
        in_specs=[
            pl.BlockSpec((_BM, _D), lambda i: (i, 0)),
            pl.BlockSpec((_D, _D), lambda i: (0, 0)),
            pl.BlockSpec((_BM, 1), lambda i: (i, 0)),
            pl.BlockSpec((1, _D), lambda i: (0, 0)),
            pl.BlockSpec((1, _D), lambda i: (0, 0)),
        ],
        out_specs=pl.BlockSpec((_BM, _D), lambda i: (i, 0)),
        out_shape=jax.ShapeDtypeStruct((_N, _D), jnp.float32),
    )(x, W, dinv_col, a_row, c_row)


def _post_call(accpair, y, dinv_col, b_row):
    def kern(acc_ref, y_ref, d_ref, b_ref, r_ref, s_ref, q_ref):
        i = pl.program_id(0)
        z = d_ref[...] * (acc_ref[0] + acc_ref[1] + y_ref[...]) + b_ref[...]
        r = jnp.maximum(z, 0.0)
        r_ref[...] = r

        @pl.when(i == 0)
        def _():
            s_ref[...] = jnp.zeros_like(s_ref)
            q_ref[...] = jnp.zeros_like(q_ref)

        s_ref[...] += jnp.sum(r, axis=0, keepdims=True)
        q_ref[...] += jnp.sum(r * r, axis=0, keepdims=True)

    return pl.pallas_call(
        kern,
        grid=(_N // _BM,),
        in_specs=[
            pl.BlockSpec((_NC, _BM, _D), lambda i: (0, i, 0)),
            pl.BlockSpec((_BM, _D), lambda i: (i, 0)),
            pl.BlockSpec((_BM, 1), lambda i: (i, 0)),
            pl.BlockSpec((1, _D), lambda i: (0, 0)),
        ],
        out_specs=[
            pl.BlockSpec((_BM, _D), lambda i: (i, 0)),
            pl.BlockSpec((1, _D), lambda i: (0, 0)),
            pl.BlockSpec((1, _D), lambda i: (0, 0)),
        ],
        out_shape=[
            jax.ShapeDtypeStruct((_N, _D), jnp.float32),
            jax.ShapeDtypeStruct((1, _D), jnp.float32),
            jax.ShapeDtypeStruct((1, _D), jnp.float32),
        ],
    )(accpair, y, dinv_col, b_row)


def _final_call(partials, gexpr_p, Wc1_p, bc1r, gc1r, bec1r, Wc2, bc2r):
    dcp = gexpr_p.shape[1]

    def kern(p_ref, g_ref, w1_ref, b1_ref, g1_ref, be1_ref, w2_ref, b2_ref,
             xd_ref, xc_ref):
        xd_ref[...] = jnp.max(p_ref[...], axis=0)
        t = jnp.tanh(jnp.dot(g_ref[...], w1_ref[...],
                             preferred_element_type=jnp.float32) + b1_ref[...])
        m = jnp.mean(t, axis=0, keepdims=True)
        v = jnp.mean(t * t, axis=0, keepdims=True) - m * m
        tn = (t - m) * lax.rsqrt(v + 1e-5) * g1_ref[...] + be1_ref[...]
        xc_ref[...] = jnp.maximum(
            jnp.dot(tn, w2_ref[...], preferred_element_type=jnp.float32)
            + b2_ref[...], 0.0)

    return pl.pallas_call(
        kern,
        out_shape=[
            jax.ShapeDtypeStruct((_B, _D), jnp.float32),
            jax.ShapeDtypeStruct((_B, _D), jnp.float32),
        ],
    )(partials, gexpr_p, Wc1_p, bc1r, gc1r, bec1r, Wc2, bc2r)


def kernel(drug_feature, drug_adj, ibatch, gexpr_data,
           W1, b1, g1, be1, W2, b2, g2, be2,
           Wc1, bc1, gc1, bec1, Wc2, bc2):
    src3 = drug_adj[0].reshape(_NW, _NWIN, _WIN)
    dst3 = drug_adj[1].reshape(_NW, _NWIN, _WIN)

    degp = _deg_call(dst3).reshape(_NC, _N)
    dinv = lax.rsqrt(degp[0] + degp[1] + 1.0)[:, None]

    ones_r = jnp.ones((1, _D), jnp.float32)
    zeros_r = jnp.zeros((1, _D), jnp.float32)

    y1 = _mm_scaled(drug_feature, W1, dinv, ones_r, zeros_r)
    acc1 = _scatter_call(src3, dst3, y1)
    r1, s1, q1 = _post_call(acc1, y1, dinv, b1.reshape(1, _D))
    m1 = s1 / _N
    v1 = q1 / _N - m1 * m1
    a1 = g1.reshape(1, _D) * lax.rsqrt(v1 + 1e-5)
    c1 = be1.reshape(1, _D) - m1 * a1

    y2 = _mm_scaled(r1, W2, dinv, a1, c1)
    acc2 = _scatter_call(src3, dst3, y2)
    r2, s2, q2 = _post_call(acc2, y2, dinv, b2.reshape(1, _D))
    m2 = s2 / _N
    v2 = q2 / _N - m2 * m2
    a2 = (g2.reshape(1, _D) * lax.rsqrt(v2 + 1e-5)).reshape(_D)
    c2 = (be2.reshape(1, _D) - m2 * (g2.reshape(1, _D) * lax.rsqrt(v2 + 1e-5)
                                     )).reshape(_D)

    partials = _segmax_call(r2, ibatch, a2, c2)

    gexpr_p = jnp.pad(gexpr_data, ((0, 0), (0, 7)))
    Wc1_p = jnp.pad(Wc1, ((0, 7), (0, 0)))
    x_drug, xc = _final_call(
        partials, gexpr_p, Wc1_p, bc1.reshape(1, _D), gc1.reshape(1, _D),
        bec1.reshape(1, _D), Wc2, bc2.reshape(1, _D))
    return (x_drug, xc)


# speedup vs baseline: 27.0502x; 1.4655x over previous
import functools

import jax
import jax.numpy as jnp
from jax import lax
from jax.experimental import pallas as pl
from jax.experimental.pallas import tpu as pltpu
from jax.experimental.pallas import tpu_sc as plsc

_N = 10000
_E = 320000
_D = 128
_B = 128
_NC = 2
_NS = 16
_NW = _NC * _NS
_EPT = _E // _NW
_WIN = 80
_NWIN = _EPT // _WIN
_RPS = _N // _NS
_ZCH = 624
_SEG = 320
_RING = 2
_IH = 64
_BM = 1000


def _sc_mesh():
    return plsc.VectorSubcoreMesh(
        core_axis_name="c", subcore_axis_name="s",
        num_cores=_NC, num_subcores=_NS)


def _deg_call(dst3):
    @functools.partial(
        pl.kernel,
        out_type=jax.ShapeDtypeStruct((_NC * _N,), jnp.float32),
        mesh=_sc_mesh(),
        scratch_types=[
            pltpu.VMEM((_NWIN, _WIN), jnp.int32),
            pltpu.VMEM((_WIN,), jnp.float32),
            pltpu.VMEM((_ZCH,), jnp.float32),
            pltpu.VMEM_SHARED((_N,), jnp.float32),
            pltpu.SemaphoreType.DMA,
        ],
    )
    def deg_kernel(dst_hbm, out_hbm, idx_v, ones_v, z_v, deg_sh, sem):
        c = lax.axis_index("c")
        s = lax.axis_index("s")
        w = s * _NC + c

        def fill_ones(i, _):
            ones_v[pl.ds(i * 16, 16)] = jnp.full((16,), 1.0, jnp.float32)
            return 0
        lax.fori_loop(0, _WIN // 16, fill_ones, 0)

        def fill_zeros(i, _):
            z_v[pl.ds(i * 16, 16)] = jnp.zeros((16,), jnp.float32)
            return 0
        lax.fori_loop(0, _ZCH // 16, fill_zeros, 0)

        pltpu.sync_copy(z_v, deg_sh.at[pl.ds(s * _ZCH, _ZCH)])

        @pl.when(s == 0)
        def _():
            pltpu.sync_copy(z_v.at[pl.ds(0, 16)],
                            deg_sh.at[pl.ds(_NS * _ZCH, _N - _NS * _ZCH)])
        plsc.subcore_barrier()

        pltpu.sync_copy(dst_hbm.at[w], idx_v)

        def win(i, _):
            pltpu.sync_copy(ones_v, deg_sh.at[idx_v.at[i]], add=True)
            return 0
        lax.fori_loop(0, _NWIN, win, 0)
        plsc.subcore_barrier()

        pltpu.sync_copy(deg_sh.at[pl.ds(s * _ZCH, _ZCH)], z_v)
        pltpu.sync_copy(z_v, out_hbm.at[pl.ds(c * _N + s * _ZCH, _ZCH)])

        @pl.when(s == 0)
        def _():
            tail = _N - _NS * _ZCH
            pltpu.sync_copy(deg_sh.at[pl.ds(_NS * _ZCH, tail)],
                            z_v.at[pl.ds(0, tail)])
            pltpu.sync_copy(z_v.at[pl.ds(0, tail)],
                            out_hbm.at[pl.ds(c * _N + _NS * _ZCH, tail)])

    return deg_kernel(dst3)


def _scatter_call(src3, dst3, y):
    @functools.partial(
        pl.kernel,
        out_type=jax.ShapeDtypeStruct((_NC, _N, _D), jnp.float32),
        mesh=_sc_mesh(),
        scratch_types=[
            pltpu.VMEM((_IH, _WIN), jnp.int32),
            pltpu.VMEM((_IH, _WIN), jnp.int32),
        ] + [pltpu.VMEM((_WIN, _D), jnp.float32) for _ in range(_RING)] + [
            pltpu.VMEM_SHARED((_N, _D), jnp.float32),
        ] + [pltpu.SemaphoreType.DMA for _ in range(_RING)],
    )
    def scat_kernel(src_hbm, dst_hbm, y_hbm, out_hbm,
                    sidx_v, didx_v, *rest):
        ring = rest[:_RING]
        acc_sh = rest[_RING]
        gsems = rest[_RING + 1:]
        r0 = ring[0]
        c = lax.axis_index("c")
        s = lax.axis_index("s")
        w = s * _NC + c
        base = s * _ZCH
        chunks = [(base + k * _WIN, _WIN) for k in range(_ZCH // _WIN)]
        rem = _ZCH - (_ZCH // _WIN) * _WIN
        if rem:
            chunks.append((base + (_ZCH // _WIN) * _WIN, rem))

        z_v = r0

        def zrow(i, _):
            for j in range(_D // 16):
                r0[i, pl.ds(j * 16, 16)] = jnp.zeros((16,), jnp.float32)
            return 0
        lax.fori_loop(0, _WIN, zrow, 0)
        for off, ln in chunks:
            pltpu.sync_copy(z_v.at[pl.ds(0, ln)],
                            acc_sh.at[pl.ds(off, ln)])

        tail_off = _NS * _ZCH
        tail = _N - tail_off

        @pl.when(s == 0)
        def _():
            pltpu.sync_copy(z_v.at[pl.ds(0, tail)],
                            acc_sh.at[pl.ds(tail_off, tail)])
        plsc.subcore_barrier()

        pltpu.sync_copy(src_hbm.at[w, pl.ds(0, _IH)], sidx_v)
        pltpu.sync_copy(dst_hbm.at[w, pl.ds(0, _IH)], didx_v)

        def _row(i):
            return jnp.where(i >= _IH, i - _IH, i)

        for j in range(_RING):
            pltpu.async_copy(y_hbm.at[sidx_v.at[j]], ring[j], gsems[j])

        nmain = (_NWIN // _RING) * _RING

        def win_ring(k, _):
            for j in range(_RING):
                i = k * _RING + j
                slot = ring[j]

                pltpu.make_async_copy(
                    y_hbm.at[sidx_v.at[_row(i)]], slot, gsems[j]).wait()
                pltpu.sync_copy(slot, acc_sh.at[didx_v.at[_row(i)]],
                                add=True)

                @pl.when(i == _IH - _RING)
                def _(i=i):
                    nh = _NWIN - _IH
                    pltpu.sync_copy(src_hbm.at[w, pl.ds(_IH, nh)],
                                    sidx_v.at[pl.ds(0, nh)])
                    pltpu.sync_copy(dst_hbm.at[w, pl.ds(_IH, nh)],
                                    didx_v.at[pl.ds(0, nh)])

                @pl.when(i + _RING < _NWIN)
                def _(i=i, slot=slot, j=j):
                    pltpu.async_copy(
                        y_hbm.at[sidx_v.at[_row(i + _RING)]], slot, gsems[j])
            return 0
        lax.fori_loop(0, nmain // _RING, win_ring, 0)
        for i in range(nmain, _NWIN):
            j = i % _RING
            pltpu.make_async_copy(
                y_hbm.at[sidx_v.at[_row(i)]], ring[j], gsems[j]).wait()
            pltpu.sync_copy(ring[j], acc_sh.at[didx_v.at[_row(i)]], add=True)
        plsc.subcore_barrier()

        for off, ln in chunks:
            pltpu.sync_copy(acc_sh.at[pl.ds(off, ln)],
                            z_v.at[pl.ds(0, ln)])
            pltpu.sync_copy(z_v.at[pl.ds(0, ln)],
                            out_hbm.at[c, pl.ds(off, ln)])

        @pl.when(s == 0)
        def _():
            pltpu.sync_copy(acc_sh.at[pl.ds(tail_off, tail)],
                            z_v.at[pl.ds(0, tail)])
            pltpu.sync_copy(z_v.at[pl.ds(0, tail)],
                            out_hbm.at[c, pl.ds(tail_off, tail)])

    return scat_kernel(src3, dst3, y)


def _segmax_call(r2, ibatch, a2, c2):
    @functools.partial(
        pl.kernel,
        out_type=jax.ShapeDtypeStruct((_NW, _B, _D), jnp.float32),
        mesh=_sc_mesh(),
        scratch_types=[
            pltpu.VMEM((_SEG, _D), jnp.float32),
            pltpu.VMEM((_SEG,), jnp.int32),
            pltpu.VMEM((_B, _D), jnp.float32),
            pltpu.VMEM((_D,), jnp.float32),
            pltpu.VMEM((_D,), jnp.float32),
            pltpu.SemaphoreType.DMA,
        ],
    )
    def seg_kernel(r_hbm, ib_hbm, a_hbm, c_hbm, out_hbm,
                   rows_v, seg_v, out_v, a_v, c_v, sem):
        c = lax.axis_index("c")
        s = lax.axis_index("s")
        w = s * _NC + c
        start = jnp.minimum(w * _SEG, _N - _SEG)

        pltpu.sync_copy(a_hbm, a_v)
        pltpu.sync_copy(c_hbm, c_v)
        pltpu.sync_copy(r_hbm.at[pl.ds(start, _SEG)], rows_v)
        pltpu.sync_copy(ib_hbm.at[pl.ds(start, _SEG)], seg_v)

        neg = jnp.full((16,), -jnp.inf, jnp.float32)

        def initrow(i, _):
            for j in range(_D // 16):
                out_v[i, pl.ds(j * 16, 16)] = neg
            return 0
        lax.fori_loop(0, _B, initrow, 0)

        def row16(k, _):
            base16 = k * 16
            seg16 = seg_v[pl.ds(base16, 16)]
            for r in range(16):
                seg = seg16[r]
                for j in range(_D // 16):
                    sl = pl.ds(j * 16, 16)
                    val = rows_v[base16 + r, sl] * a_v[sl] + c_v[sl]
                    out_v[seg, sl] = jnp.maximum(out_v[seg, sl], val)
            return 0
        lax.fori_loop(0, _SEG // 16, row16, 0)

        pltpu.sync_copy(out_v, out_hbm.at[w])

    return seg_kernel(r2, ibatch, a2, c2)


def _mm_scaled(x, W, dinv_col, a_row, c_row):
    def kern(x_ref, w_ref, d_ref, a_ref, c_ref, o_ref):
        xb = x_ref[...] * a_ref[...] + c_ref[...]
        o_ref[...] = d_ref[...] * jnp.dot(
            xb, w_ref[...], preferred_element_type=jnp.float32)

    return pl.pallas_call(
        kern,
        grid=(_N // _BM,),
        in_specs=[
            pl.BlockSpec((_BM, _D), lambda i: (i, 0)),
            pl.BlockSpec((_D, _D), lambda i: (0, 0)),
            pl.BlockSpec((_BM, 1), lambda i: (i, 0)),
            pl.BlockSpec((1, _D), lambda i: (0, 0)),
            pl.BlockSpec((1, _D), lambda i: (0, 0)),
        ],
        out_specs=pl.BlockSpec((_BM, _D), lambda i: (i, 0)),
        out_shape=jax.ShapeDtypeStruct((_N, _D), jnp.float32),
    )(x, W, dinv_col, a_row, c_row)


def _post_call(accpair, y, dinv_col, b_row):
    def kern(acc_ref, y_ref, d_ref, b_ref, r_ref, s_ref, q_ref):
        i = pl.program_id(0)
        z = d_ref[...] * (acc_ref[0] + acc_ref[1] + y_ref[...]) + b_ref[...]
        r = jnp.maximum(z, 0.0)
        r_ref[...] = r

        @pl.when(i == 0)
        def _():
            s_ref[...] = jnp.zeros_like(s_ref)
            q_ref[...] = jnp.zeros_like(q_ref)

        s_ref[...] += jnp.sum(r, axis=0, keepdims=True)
        q_ref[...] += jnp.sum(r * r, axis=0, keepdims=True)

    return pl.pallas_call(
        kern,
        grid=(_N // _BM,),
        in_specs=[
            pl.BlockSpec((_NC, _BM, _D), lambda i: (0, i, 0)),
            pl.BlockSpec((_BM, _D), lambda i: (i, 0)),
            pl.BlockSpec((_BM, 1), lambda i: (i, 0)),
            pl.BlockSpec((1, _D), lambda i: (0, 0)),
        ],
        out_specs=[
            pl.BlockSpec((_BM, _D), lambda i: (i, 0)),
            pl.BlockSpec((1, _D), lambda i: (0, 0)),
            pl.BlockSpec((1, _D), lambda i: (0, 0)),
        ],
        out_shape=[
            jax.ShapeDtypeStruct((_N, _D), jnp.float32),
            jax.ShapeDtypeStruct((1, _D), jnp.float32),
            jax.ShapeDtypeStruct((1, _D), jnp.float32),
        ],
    )(accpair, y, dinv_col, b_row)


def _final_call(partials, gexpr_p, Wc1_p, bc1r, gc1r, bec1r, Wc2, bc2r):
    dcp = gexpr_p.shape[1]

    def kern(p_ref, g_ref, w1_ref, b1_ref, g1_ref, be1_ref, w2_ref, b2_ref,
             xd_ref, xc_ref):
        xd_ref[...] = jnp.max(p_ref[...], axis=0)
        t = jnp.tanh(jnp.dot(g_ref[...], w1_ref[...],
                             preferred_element_type=jnp.float32) + b1_ref[...])
        m = jnp.mean(t, axis=0, keepdims=True)
        v = jnp.mean(t * t, axis=0, keepdims=True) - m * m
        tn = (t - m) * lax.rsqrt(v + 1e-5) * g1_ref[...] + be1_ref[...]
        xc_ref[...] = jnp.maximum(
            jnp.dot(tn, w2_ref[...], preferred_element_type=jnp.float32)
            + b2_ref[...], 0.0)

    return pl.pallas_call(
        kern,
        out_shape=[
            jax.ShapeDtypeStruct((_B, _D), jnp.float32),
            jax.ShapeDtypeStruct((_B, _D), jnp.float32),
        ],
    )(partials, gexpr_p, Wc1_p, bc1r, gc1r, bec1r, Wc2, bc2r)


def kernel(drug_feature, drug_adj, ibatch, gexpr_data,
           W1, b1, g1, be1, W2, b2, g2, be2,
           Wc1, bc1, gc1, bec1, Wc2, bc2):
    src3 = drug_adj[0].reshape(_NW, _NWIN, _WIN)
    dst3 = drug_adj[1].reshape(_NW, _NWIN, _WIN)

    degp = _deg_call(dst3).reshape(_NC, _N)
    dinv = lax.rsqrt(degp[0] + degp[1] + 1.0)[:, None]

    ones_r = jnp.ones((1, _D), jnp.float32)
    zeros_r = jnp.zeros((1, _D), jnp.float32)

    y1 = _mm_scaled(drug_feature, W1, dinv, ones_r, zeros_r)
    acc1 = _scatter_call(src3, dst3, y1)
    r1, s1, q1 = _post_call(acc1, y1, dinv, b1.reshape(1, _D))
    m1 = s1 / _N
    v1 = q1 / _N - m1 * m1
    a1 = g1.reshape(1, _D) * lax.rsqrt(v1 + 1e-5)
    c1 = be1.reshape(1, _D) - m1 * a1

    y2 = _mm_scaled(r1, W2, dinv, a1, c1)
    acc2 = _scatter_call(src3, dst3, y2)
    r2, s2, q2 = _post_call(acc2, y2, dinv, b2.reshape(1, _D))
    m2 = s2 / _N
    v2 = q2 / _N - m2 * m2
    a2 = (g2.reshape(1, _D) * lax.rsqrt(v2 + 1e-5)).reshape(_D)
    c2 = (be2.reshape(1, _D) - m2 * (g2.reshape(1, _D) * lax.rsqrt(v2 + 1e-5)
                                     )).reshape(_D)

    partials = _segmax_call(r2, ibatch, a2, c2)

    gexpr_p = jnp.pad(gexpr_data, ((0, 0), (0, 7)))
    Wc1_p = jnp.pad(Wc1, ((0, 7), (0, 0)))
    x_drug, xc = _final_call(
        partials, gexpr_p, Wc1_p, bc1.reshape(1, _D), gc1.reshape(1, _D),
        bec1.reshape(1, _D), Wc2, bc2.reshape(1, _D))
    return (x_drug, xc)


# speedup vs baseline: 30.3751x; 1.1229x over previous
import functools

import jax
import jax.numpy as jnp
from jax import lax
from jax.experimental import pallas as pl
from jax.experimental.pallas import tpu as pltpu
from jax.experimental.pallas import tpu_sc as plsc

_N = 10000
_E = 320000
_D = 128
_B = 128
_NC = 2
_NS = 16
_NW = _NC * _NS
_EPT = _E // _NW
_WIN = 80
_NWIN = _EPT // _WIN
_RPS = _N // _NS
_ZCH = 624
_SEG = 320
_R = 3
_Q = 6
_NWIN2 = 126
_NACC = _N + 16
_BM = 1000


def _sc_mesh():
    return plsc.VectorSubcoreMesh(
        core_axis_name="c", subcore_axis_name="s",
        num_cores=_NC, num_subcores=_NS)


def _deg_call(dst3):
    @functools.partial(
        pl.kernel,
        out_type=jax.ShapeDtypeStruct((_NC * _N,), jnp.float32),
        mesh=_sc_mesh(),
        scratch_types=[
            pltpu.VMEM((_NWIN, _WIN), jnp.int32),
            pltpu.VMEM((_WIN,), jnp.float32),
            pltpu.VMEM((_ZCH,), jnp.float32),
            pltpu.VMEM_SHARED((_N,), jnp.float32),
            pltpu.SemaphoreType.DMA,
        ],
    )
    def deg_kernel(dst_hbm, out_hbm, idx_v, ones_v, z_v, deg_sh, sem):
        c = lax.axis_index("c")
        s = lax.axis_index("s")
        w = s * _NC + c

        def fill_ones(i, _):
            ones_v[pl.ds(i * 16, 16)] = jnp.full((16,), 1.0, jnp.float32)
            return 0
        lax.fori_loop(0, _WIN // 16, fill_ones, 0)

        def fill_zeros(i, _):
            z_v[pl.ds(i * 16, 16)] = jnp.zeros((16,), jnp.float32)
            return 0
        lax.fori_loop(0, _ZCH // 16, fill_zeros, 0)

        pltpu.sync_copy(z_v, deg_sh.at[pl.ds(s * _ZCH, _ZCH)])

        @pl.when(s == 0)
        def _():
            pltpu.sync_copy(z_v.at[pl.ds(0, 16)],
                            deg_sh.at[pl.ds(_NS * _ZCH, _N - _NS * _ZCH)])
        plsc.subcore_barrier()

        pltpu.sync_copy(dst_hbm.at[w], idx_v)

        def win(i, _):
            pltpu.sync_copy(ones_v, deg_sh.at[idx_v.at[i]], add=True)
            return 0
        lax.fori_loop(0, _NWIN, win, 0)
        plsc.subcore_barrier()

        pltpu.sync_copy(deg_sh.at[pl.ds(s * _ZCH, _ZCH)], z_v)
        pltpu.sync_copy(z_v, out_hbm.at[pl.ds(c * _N + s * _ZCH, _ZCH)])

        @pl.when(s == 0)
        def _():
            tail = _N - _NS * _ZCH
            pltpu.sync_copy(deg_sh.at[pl.ds(_NS * _ZCH, tail)],
                            z_v.at[pl.ds(0, tail)])
            pltpu.sync_copy(z_v.at[pl.ds(0, tail)],
                            out_hbm.at[pl.ds(c * _N + _NS * _ZCH, tail)])

    return deg_kernel(dst3)


def _scatter_call(idx_flat, y):
    @functools.partial(
        pl.kernel,
        out_type=jax.ShapeDtypeStruct((_NC, _N, _D), jnp.float32),
        mesh=_sc_mesh(),
        scratch_types=[
            pltpu.VMEM((_Q, 2, _WIN), jnp.int32),
        ] + [pltpu.VMEM((_WIN, _D), jnp.float32) for _ in range(_R)]
          + [pltpu.VMEM_SHARED((_NACC, _D), jnp.float32)]
          + [pltpu.SemaphoreType.DMA] * (2 * _R + _Q),
    )
    def scat_kernel(idx_hbm, y_hbm, out_hbm, idxb, *rest):
        rows = rest[:_R]
        acc_sh = rest[_R]
        sems = rest[_R + 1:]
        gsem = sems[:_R]
        ssem = sems[_R:2 * _R]
        isem = sems[2 * _R:]
        z_v = rows[0]
        c = lax.axis_index("c")
        s = lax.axis_index("s")
        w = s * _NC + c
        tb = w * _NWIN2
        base = s * _ZCH
        chunks = [(base + k * _WIN, _WIN) for k in range(_ZCH // _WIN)]
        rem = _ZCH - (_ZCH // _WIN) * _WIN
        if rem:
            chunks.append((base + (_ZCH // _WIN) * _WIN, rem))

        def zrow(i, _):
            for j in range(_D // 16):
                z_v[i, pl.ds(j * 16, 16)] = jnp.zeros((16,), jnp.float32)
            return 0
        lax.fori_loop(0, _WIN, zrow, 0)
        for off, ln in chunks:
            pltpu.sync_copy(z_v.at[pl.ds(0, ln)], acc_sh.at[pl.ds(off, ln)])

        tail_off = _NS * _ZCH
        tail = _N - tail_off

        @pl.when(s == 0)
        def _():
            pltpu.sync_copy(z_v.at[pl.ds(0, tail)],
                            acc_sh.at[pl.ds(tail_off, tail)])
        plsc.subcore_barrier()

        def idx_load(i, q):
            pltpu.async_copy(idx_hbm.at[tb + i], idxb.at[q], isem[q])

        def idx_wait(q):
            pltpu.make_async_copy(idx_hbm.at[tb], idxb.at[q],
                                  isem[q]).wait()

        def gather(i, j, q):
            pltpu.async_copy(y_hbm.at[idxb.at[q, 0]], rows[j], gsem[j])

        def gather_wait(j, q):
            pltpu.make_async_copy(y_hbm.at[idxb.at[q, 0]], rows[j],
                                  gsem[j]).wait()

        def scatter(j, q):
            pltpu.async_copy(rows[j], acc_sh.at[idxb.at[q, 1]], ssem[j],
                             add=True)

        def scatter_wait(j):
            pltpu.make_async_copy(rows[j], acc_sh.at[idxb.at[0, 1]],
                                  ssem[j]).wait()

        for i in range(4):
            idx_load(i, i)
        for i in range(2):
            idx_wait(i)
            gather(i, i, i)

        def win6(k, _):
            for jj in range(6):
                i = k * 6 + jj
                j = jj % _R
                q = jj % _Q
                jn = (jj + 2) % _R

                @pl.when(i >= 1)
                def _(jn=jn):
                    scatter_wait(jn)
                gather_wait(j, q)
                scatter(j, q)

                @pl.when(i + 4 < _NWIN2)
                def _(i=i, qn=(jj + 4) % _Q):
                    idx_load(i + 4, qn)

                @pl.when(i + 2 < _NWIN2)
                def _(i=i, jn=jn, qn2=(jj + 2) % _Q):
                    idx_wait(qn2)
                    gather(i + 2, jn, qn2)
            return 0
        lax.fori_loop(0, _NWIN2 // 6, win6, 0)
        scatter_wait((_NWIN2 - 1) % _R)
        plsc.subcore_barrier()

        for off, ln in chunks:
            pltpu.sync_copy(acc_sh.at[pl.ds(off, ln)], z_v.at[pl.ds(0, ln)])
            pltpu.sync_copy(z_v.at[pl.ds(0, ln)],
                            out_hbm.at[c, pl.ds(off, ln)])

        @pl.when(s == 0)
        def _():
            pltpu.sync_copy(acc_sh.at[pl.ds(tail_off, tail)],
                            z_v.at[pl.ds(0, tail)])
            pltpu.sync_copy(z_v.at[pl.ds(0, tail)],
                            out_hbm.at[c, pl.ds(tail_off, tail)])

    return scat_kernel(idx_flat, y)


def _segmax_call(r2, ibatch, a2, c2):
    @functools.partial(
        pl.kernel,
        out_type=jax.ShapeDtypeStruct((_NW, _B, _D), jnp.float32),
        mesh=_sc_mesh(),
        scratch_types=[
            pltpu.VMEM((_SEG, _D), jnp.float32),
            pltpu.VMEM((_SEG,), jnp.int32),
            pltpu.VMEM((_B, _D), jnp.float32),
            pltpu.VMEM((_D,), jnp.float32),
            pltpu.VMEM((_D,), jnp.float32),
            pltpu.SemaphoreType.DMA,
        ],
    )
    def seg_kernel(r_hbm, ib_hbm, a_hbm, c_hbm, out_hbm,
                   rows_v, seg_v, out_v, a_v, c_v, sem):
        c = lax.axis_index("c")
        s = lax.axis_index("s")
        w = s * _NC + c
        start = jnp.minimum(w * _SEG, _N - _SEG)

        pltpu.sync_copy(a_hbm, a_v)
        pltpu.sync_copy(c_hbm, c_v)
        pltpu.sync_copy(r_hbm.at[pl.ds(start, _SEG)], rows_v)
        pltpu.sync_copy(ib_hbm.at[pl.ds(start, _SEG)], seg_v)

        neg = jnp.full((16,), -jnp.inf, jnp.float32)

        def initrow(i, _):
            for j in range(_D // 16):
                out_v[i, pl.ds(j * 16, 16)] = neg
            return 0
        lax.fori_loop(0, _B, initrow, 0)

        def row16(k, _):
            base16 = k * 16
            seg16 = seg_v[pl.ds(base16, 16)]
            for r in range(16):
                seg = seg16[r]
                for j in range(_D // 16):
                    sl = pl.ds(j * 16, 16)
                    val = rows_v[base16 + r, sl] * a_v[sl] + c_v[sl]
                    out_v[seg, sl] = jnp.maximum(out_v[seg, sl], val)
            return 0
        lax.fori_loop(0, _SEG // 16, row16, 0)

        pltpu.sync_copy(out_v, out_hbm.at[w])

    return seg_kernel(r2, ibatch, a2, c2)


def _mm_scaled(x, W, dinv_col, a_row, c_row):
    def kern(x_ref, w_ref, d_ref, a_ref, c_ref, o_ref):
        xb = x_ref[...] * a_ref[...] + c_ref[...]
        o_ref[...] = d_ref[...] * jnp.dot(
            xb, w_ref[...], preferred_element_type=jnp.float32)

    return pl.pallas_call(
        kern,
        grid=(_N // _BM,),
        in_specs=[
            pl.BlockSpec((_BM, _D), lambda i: (i, 0)),
            pl.BlockSpec((_D, _D), lambda i: (0, 0)),
            pl.BlockSpec((_BM, 1), lambda i: (i, 0)),
            pl.BlockSpec((1, _D), lambda i: (0, 0)),
            pl.BlockSpec((1, _D), lambda i: (0, 0)),
        ],
        out_specs=pl.BlockSpec((_BM, _D), lambda i: (i, 0)),
        out_shape=jax.ShapeDtypeStruct((_N, _D), jnp.float32),
    )(x, W, dinv_col, a_row, c_row)


def _post_call(accpair, y, dinv_col, b_row):
    def kern(acc_ref, y_ref, d_ref, b_ref, r_ref, s_ref, q_ref):
        i = pl.program_id(0)
        z = d_ref[...] * (acc_ref[0] + acc_ref[1] + y_ref[...]) + b_ref[...]
        r = jnp.maximum(z, 0.0)
        r_ref[...] = r

        @pl.when(i == 0)
        def _():
            s_ref[...] = jnp.zeros_like(s_ref)
            q_ref[...] = jnp.zeros_like(q_ref)

        s_ref[...] += jnp.sum(r, axis=0, keepdims=True)
        q_ref[...] += jnp.sum(r * r, axis=0, keepdims=True)

    return pl.pallas_call(
        kern,
        grid=(_N // _BM,),
        in_specs=[
            pl.BlockSpec((_NC, _BM, _D), lambda i: (0, i, 0)),
            pl.BlockSpec((_BM, _D), lambda i: (i, 0)),
            pl.BlockSpec((_BM, 1), lambda i: (i, 0)),
            pl.BlockSpec((1, _D), lambda i: (0, 0)),
        ],
        out_specs=[
            pl.BlockSpec((_BM, _D), lambda i: (i, 0)),
            pl.BlockSpec((1, _D), lambda i: (0, 0)),
            pl.BlockSpec((1, _D), lambda i: (0, 0)),
        ],
        out_shape=[
            jax.ShapeDtypeStruct((_N, _D), jnp.float32),
            jax.ShapeDtypeStruct((1, _D), jnp.float32),
            jax.ShapeDtypeStruct((1, _D), jnp.float32),
        ],
    )(accpair, y, dinv_col, b_row)


def _final_call(partials, gexpr_p, Wc1_p, bc1r, gc1r, bec1r, Wc2, bc2r):
    dcp = gexpr_p.shape[1]

    def kern(p_ref, g_ref, w1_ref, b1_ref, g1_ref, be1_ref, w2_ref, b2_ref,
             xd_ref, xc_ref):
        xd_ref[...] = jnp.max(p_ref[...], axis=0)
        t = jnp.tanh(jnp.dot(g_ref[...], w1_ref[...],
                             preferred_element_type=jnp.float32) + b1_ref[...])
        m = jnp.mean(t, axis=0, keepdims=True)
        v = jnp.mean(t * t, axis=0, keepdims=True) - m * m
        tn = (t - m) * lax.rsqrt(v + 1e-5) * g1_ref[...] + be1_ref[...]
        xc_ref[...] = jnp.maximum(
            jnp.dot(tn, w2_ref[...], preferred_element_type=jnp.float32)
            + b2_ref[...], 0.0)

    return pl.pallas_call(
        kern,
        out_shape=[
            jax.ShapeDtypeStruct((_B, _D), jnp.float32),
            jax.ShapeDtypeStruct((_B, _D), jnp.float32),
        ],
    )(partials, gexpr_p, Wc1_p, bc1r, gc1r, bec1r, Wc2, bc2r)


def kernel(drug_feature, drug_adj, ibatch, gexpr_data,
           W1, b1, g1, be1, W2, b2, g2, be2,
           Wc1, bc1, gc1, bec1, Wc2, bc2):
    src = drug_adj[0]
    dst = drug_adj[1]
    dst3 = dst.reshape(_NW, _NWIN, _WIN)
    pk = jnp.arange(_WIN, dtype=jnp.int32)
    pad_src = jnp.broadcast_to(pk * 125, (_NW, _WIN))
    pad_dst = jnp.broadcast_to(_N + (pk % 16), (_NW, _WIN))
    srcp = jnp.concatenate([src.reshape(_NW, _EPT), pad_src], axis=1)
    dstp = jnp.concatenate([dst.reshape(_NW, _EPT), pad_dst], axis=1)
    idx_flat = jnp.stack([srcp.reshape(_NW, _NWIN2, _WIN),
                          dstp.reshape(_NW, _NWIN2, _WIN)],
                         axis=2).reshape(_NW * _NWIN2, 2, _WIN)

    degp = _deg_call(dst3).reshape(_NC, _N)
    dinv = lax.rsqrt(degp[0] + degp[1] + 1.0)[:, None]

    ones_r = jnp.ones((1, _D), jnp.float32)
    zeros_r = jnp.zeros((1, _D), jnp.float32)

    y1 = _mm_scaled(drug_feature, W1, dinv, ones_r, zeros_r)
    acc1 = _scatter_call(idx_flat, y1)
    r1, s1, q1 = _post_call(acc1, y1, dinv, b1.reshape(1, _D))
    m1 = s1 / _N
    v1 = q1 / _N - m1 * m1
    a1 = g1.reshape(1, _D) * lax.rsqrt(v1 + 1e-5)
    c1 = be1.reshape(1, _D) - m1 * a1

    y2 = _mm_scaled(r1, W2, dinv, a1, c1)
    acc2 = _scatter_call(idx_flat, y2)
    r2, s2, q2 = _post_call(acc2, y2, dinv, b2.reshape(1, _D))
    m2 = s2 / _N
    v2 = q2 / _N - m2 * m2
    a2 = (g2.reshape(1, _D) * lax.rsqrt(v2 + 1e-5)).reshape(_D)
    c2 = (be2.reshape(1, _D) - m2 * (g2.reshape(1, _D) * lax.rsqrt(v2 + 1e-5)
                                     )).reshape(_D)

    partials = _segmax_call(r2, ibatch, a2, c2)

    gexpr_p = jnp.pad(gexpr_data, ((0, 0), (0, 7)))
    Wc1_p = jnp.pad(Wc1, ((0, 7), (0, 0)))
    x_drug, xc = _final_call(
        partials, gexpr_p, Wc1_p, bc1.reshape(1, _D), gc1.reshape(1, _D),
        bec1.reshape(1, _D), Wc2, bc2.reshape(1, _D))
    return (x_drug, xc)


# speedup vs baseline: 30.8204x; 1.0147x over previous
import functools

import jax
import jax.numpy as jnp
from jax import lax
from jax.experimental import pallas as pl
from jax.experimental.pallas import tpu as pltpu
from jax.experimental.pallas import tpu_sc as plsc

_N = 10000
_E = 320000
_D = 128
_B = 128
_NC = 2
_NS = 16
_NW = _NC * _NS
_EPT = _E // _NW
_WIN = 80
_NWIN = _EPT // _WIN
_RPS = _N // _NS
_ZCH = 624
_SEG = 320
_R = 3
_Q = 6
_NWIN2 = 126
_NACC = _N + 16
_BM = 1000


def _sc_mesh():
    return plsc.VectorSubcoreMesh(
        core_axis_name="c", subcore_axis_name="s",
        num_cores=_NC, num_subcores=_NS)


def _deg_call(dst3):
    @functools.partial(
        pl.kernel,
        out_type=jax.ShapeDtypeStruct((_NC * _N,), jnp.float32),
        mesh=_sc_mesh(),
        scratch_types=[
            pltpu.VMEM((_NWIN, _WIN), jnp.int32),
            pltpu.VMEM((_WIN,), jnp.float32),
            pltpu.VMEM((_ZCH,), jnp.float32),
            pltpu.VMEM_SHARED((_N,), jnp.float32),
            pltpu.SemaphoreType.DMA,
        ],
    )
    def deg_kernel(dst_hbm, out_hbm, idx_v, ones_v, z_v, deg_sh, sem):
        c = lax.axis_index("c")
        s = lax.axis_index("s")
        w = s * _NC + c

        def fill_ones(i, _):
            ones_v[pl.ds(i * 16, 16)] = jnp.full((16,), 1.0, jnp.float32)
            return 0
        lax.fori_loop(0, _WIN // 16, fill_ones, 0)

        def fill_zeros(i, _):
            z_v[pl.ds(i * 16, 16)] = jnp.zeros((16,), jnp.float32)
            return 0
        lax.fori_loop(0, _ZCH // 16, fill_zeros, 0)

        pltpu.sync_copy(z_v, deg_sh.at[pl.ds(s * _ZCH, _ZCH)])

        @pl.when(s == 0)
        def _():
            pltpu.sync_copy(z_v.at[pl.ds(0, 16)],
                            deg_sh.at[pl.ds(_NS * _ZCH, _N - _NS * _ZCH)])
        plsc.subcore_barrier()

        pltpu.sync_copy(dst_hbm.at[w], idx_v)

        def win(i, _):
            pltpu.async_copy(ones_v, deg_sh.at[idx_v.at[i]], sem, add=True)
            return 0
        lax.fori_loop(0, _NWIN, win, 0)

        def drain(i, _):
            pltpu.make_async_copy(ones_v, deg_sh.at[idx_v.at[0]], sem).wait()
            return 0
        lax.fori_loop(0, _NWIN, drain, 0)
        plsc.subcore_barrier()

        pltpu.sync_copy(deg_sh.at[pl.ds(s * _ZCH, _ZCH)], z_v)
        pltpu.sync_copy(z_v, out_hbm.at[pl.ds(c * _N + s * _ZCH, _ZCH)])

        @pl.when(s == 0)
        def _():
            tail = _N - _NS * _ZCH
            pltpu.sync_copy(deg_sh.at[pl.ds(_NS * _ZCH, tail)],
                            z_v.at[pl.ds(0, tail)])
            pltpu.sync_copy(z_v.at[pl.ds(0, tail)],
                            out_hbm.at[pl.ds(c * _N + _NS * _ZCH, tail)])

    return deg_kernel(dst3)


def _scatter_call(idx_flat, y):
    @functools.partial(
        pl.kernel,
        out_type=jax.ShapeDtypeStruct((_NC, _N, _D), jnp.float32),
        mesh=_sc_mesh(),
        scratch_types=[
            pltpu.VMEM((_Q, 2, _WIN), jnp.int32),
        ] + [pltpu.VMEM((_WIN, _D), jnp.float32) for _ in range(_R)]
          + [pltpu.VMEM_SHARED((_NACC, _D), jnp.float32)]
          + [pltpu.SemaphoreType.DMA] * (2 * _R + _Q),
    )
    def scat_kernel(idx_hbm, y_hbm, out_hbm, idxb, *rest):
        rows = rest[:_R]
        acc_sh = rest[_R]
        sems = rest[_R + 1:]
        gsem = sems[:_R]
        ssem = sems[_R:2 * _R]
        isem = sems[2 * _R:]
        z_v = rows[0]
        c = lax.axis_index("c")
        s = lax.axis_index("s")
        w = s * _NC + c
        tb = w * _NWIN2
        base = s * _ZCH
        chunks = [(base + k * _WIN, _WIN) for k in range(_ZCH // _WIN)]
        rem = _ZCH - (_ZCH // _WIN) * _WIN
        if rem:
            chunks.append((base + (_ZCH // _WIN) * _WIN, rem))

        def zrow(i, _):
            for j in range(_D // 16):
                z_v[i, pl.ds(j * 16, 16)] = jnp.zeros((16,), jnp.float32)
            return 0
        lax.fori_loop(0, _WIN, zrow, 0)
        for off, ln in chunks:
            pltpu.sync_copy(z_v.at[pl.ds(0, ln)], acc_sh.at[pl.ds(off, ln)])

        tail_off = _NS * _ZCH
        tail = _N - tail_off

        @pl.when(s == 0)
        def _():
            pltpu.sync_copy(z_v.at[pl.ds(0, tail)],
                            acc_sh.at[pl.ds(tail_off, tail)])
        plsc.subcore_barrier()

        def idx_load(i, q):
            pltpu.async_copy(idx_hbm.at[tb + i], idxb.at[q], isem[q])

        def idx_wait(q):
            pltpu.make_async_copy(idx_hbm.at[tb], idxb.at[q],
                                  isem[q]).wait()

        def gather(i, j, q):
            pltpu.async_copy(y_hbm.at[idxb.at[q, 0]], rows[j], gsem[j])

        def gather_wait(j, q):
            pltpu.make_async_copy(y_hbm.at[idxb.at[q, 0]], rows[j],
                                  gsem[j]).wait()

        def scatter(j, q):
            pltpu.async_copy(rows[j], acc_sh.at[idxb.at[q, 1]], ssem[j],
                             add=True)

        def scatter_wait(j):
            pltpu.make_async_copy(rows[j], acc_sh.at[idxb.at[0, 1]],
                                  ssem[j]).wait()

        for i in range(4):
            idx_load(i, i)
        for i in range(2):
            idx_wait(i)
            gather(i, i, i)

        def win6(k, _):
            for jj in range(6):
                i = k * 6 + jj
                j = jj % _R
                q = jj % _Q
                jn = (jj + 2) % _R

                @pl.when(i >= 1)
                def _(jn=jn):
                    scatter_wait(jn)
                gather_wait(j, q)
                scatter(j, q)

                @pl.when(i + 4 < _NWIN2)
                def _(i=i, qn=(jj + 4) % _Q):
                    idx_load(i + 4, qn)

                @pl.when(i + 2 < _NWIN2)
                def _(i=i, jn=jn, qn2=(jj + 2) % _Q):
                    idx_wait(qn2)
                    gather(i + 2, jn, qn2)
            return 0
        lax.fori_loop(0, _NWIN2 // 6, win6, 0)
        scatter_wait((_NWIN2 - 1) % _R)
        plsc.subcore_barrier()

        for off, ln in chunks:
            pltpu.sync_copy(acc_sh.at[pl.ds(off, ln)], z_v.at[pl.ds(0, ln)])
            pltpu.sync_copy(z_v.at[pl.ds(0, ln)],
                            out_hbm.at[c, pl.ds(off, ln)])

        @pl.when(s == 0)
        def _():
            pltpu.sync_copy(acc_sh.at[pl.ds(tail_off, tail)],
                            z_v.at[pl.ds(0, tail)])
            pltpu.sync_copy(z_v.at[pl.ds(0, tail)],
                            out_hbm.at[c, pl.ds(tail_off, tail)])

    return scat_kernel(idx_flat, y)


def _segmax_call(r2, ibatch, a2, c2):
    @functools.partial(
        pl.kernel,
        out_type=jax.ShapeDtypeStruct((_NW, _B, _D), jnp.float32),
        mesh=_sc_mesh(),
        scratch_types=[
            pltpu.VMEM((_SEG, _D), jnp.float32),
            pltpu.VMEM((_SEG,), jnp.int32),
            pltpu.VMEM((_B, _D), jnp.float32),
            pltpu.VMEM((_D,), jnp.float32),
            pltpu.VMEM((_D,), jnp.float32),
            pltpu.SemaphoreType.DMA,
            pltpu.SemaphoreType.DMA,
        ],
    )
    def seg_kernel(r_hbm, ib_hbm, a_hbm, c_hbm, out_hbm,
                   rows_v, seg_v, out_v, a_v, c_v, sem, sem2):
        c = lax.axis_index("c")
        s = lax.axis_index("s")
        w = s * _NC + c
        start = jnp.minimum(w * _SEG, _N - _SEG)

        half = _SEG // 2
        h0 = pltpu.async_copy(r_hbm.at[pl.ds(start, half)],
                              rows_v.at[pl.ds(0, half)], sem)
        h1 = pltpu.async_copy(r_hbm.at[pl.ds(start + half, half)],
                              rows_v.at[pl.ds(half, half)], sem2)
        pltpu.sync_copy(a_hbm, a_v)
        pltpu.sync_copy(c_hbm, c_v)
        pltpu.sync_copy(ib_hbm.at[pl.ds(start, _SEG)], seg_v)

        neg = jnp.full((16,), -jnp.inf, jnp.float32)

        def initrow(i, _):
            for j in range(_D // 16):
                out_v[i, pl.ds(j * 16, 16)] = neg
            return 0
        lax.fori_loop(0, _B, initrow, 0)

        def row16(k, _):
            base16 = k * 16
            seg16 = seg_v[pl.ds(base16, 16)]
            for r in range(16):
                seg = seg16[r]
                for j in range(_D // 16):
                    sl = pl.ds(j * 16, 16)
                    val = rows_v[base16 + r, sl] * a_v[sl] + c_v[sl]
                    out_v[seg, sl] = jnp.maximum(out_v[seg, sl], val)
            return 0
        h0.wait()
        lax.fori_loop(0, half // 16, row16, 0)
        h1.wait()
        lax.fori_loop(half // 16, _SEG // 16, row16, 0)

        pltpu.sync_copy(out_v, out_hbm.at[w])

    return seg_kernel(r2, ibatch, a2, c2)


def _mm_scaled(x, W, dinv_col, a_row, c_row):
    def kern(x_ref, w_ref, d_ref, a_ref, c_ref, o_ref):
        xb = x_ref[...] * a_ref[...] + c_ref[...]
        o_ref[...] = d_ref[...] * jnp.dot(
            xb, w_ref[...], preferred_element_type=jnp.float32)

    return pl.pallas_call(
        kern,
        grid=(_N // _BM,),
        in_specs=[
            pl.BlockSpec((_BM, _D), lambda i: (i, 0)),
            pl.BlockSpec((_D, _D), lambda i: (0, 0)),
            pl.BlockSpec((_BM, 1), lambda i: (i, 0)),
            pl.BlockSpec((1, _D), lambda i: (0, 0)),
            pl.BlockSpec((1, _D), lambda i: (0, 0)),
        ],
        out_specs=pl.BlockSpec((_BM, _D), lambda i: (i, 0)),
        out_shape=jax.ShapeDtypeStruct((_N, _D), jnp.float32),
    )(x, W, dinv_col, a_row, c_row)


def _post_call(accpair, y, dinv_col, b_row):
    def kern(acc_ref, y_ref, d_ref, b_ref, r_ref, s_ref, q_ref):
        i = pl.program_id(0)
        z = d_ref[...] * (acc_ref[0] + acc_ref[1] + y_ref[...]) + b_ref[...]
        r = jnp.maximum(z, 0.0)
        r_ref[...] = r

        @pl.when(i == 0)
        def _():
            s_ref[...] = jnp.zeros_like(s_ref)
            q_ref[...] = jnp.zeros_like(q_ref)

        s_ref[...] += jnp.sum(r, axis=0, keepdims=True)
        q_ref[...] += jnp.sum(r * r, axis=0, keepdims=True)

    return pl.pallas_call(
        kern,
        grid=(_N // _BM,),
        in_specs=[
            pl.BlockSpec((_NC, _BM, _D), lambda i: (0, i, 0)),
            pl.BlockSpec((_BM, _D), lambda i: (i, 0)),
            pl.BlockSpec((_BM, 1), lambda i: (i, 0)),
            pl.BlockSpec((1, _D), lambda i: (0, 0)),
        ],
        out_specs=[
            pl.BlockSpec((_BM, _D), lambda i: (i, 0)),
            pl.BlockSpec((1, _D), lambda i: (0, 0)),
            pl.BlockSpec((1, _D), lambda i: (0, 0)),
        ],
        out_shape=[
            jax.ShapeDtypeStruct((_N, _D), jnp.float32),
            jax.ShapeDtypeStruct((1, _D), jnp.float32),
            jax.ShapeDtypeStruct((1, _D), jnp.float32),
        ],
    )(accpair, y, dinv_col, b_row)


def _final_call(partials, gexpr_p, Wc1_p, bc1r, gc1r, bec1r, Wc2, bc2r):
    dcp = gexpr_p.shape[1]

    def kern(p_ref, g_ref, w1_ref, b1_ref, g1_ref, be1_ref, w2_ref, b2_ref,
             xd_ref, xc_ref):
        xd_ref[...] = jnp.max(p_ref[...], axis=0)
        t = jnp.tanh(jnp.dot(g_ref[...], w1_ref[...],
                             preferred_element_type=jnp.float32) + b1_ref[...])
        m = jnp.mean(t, axis=0, keepdims=True)
        v = jnp.mean(t * t, axis=0, keepdims=True) - m * m
        tn = (t - m) * lax.rsqrt(v + 1e-5) * g1_ref[...] + be1_ref[...]
        xc_ref[...] = jnp.maximum(
            jnp.dot(tn, w2_ref[...], preferred_element_type=jnp.float32)
            + b2_ref[...], 0.0)

    return pl.pallas_call(
        kern,
        out_shape=[
            jax.ShapeDtypeStruct((_B, _D), jnp.float32),
            jax.ShapeDtypeStruct((_B, _D), jnp.float32),
        ],
    )(partials, gexpr_p, Wc1_p, bc1r, gc1r, bec1r, Wc2, bc2r)


def kernel(drug_feature, drug_adj, ibatch, gexpr_data,
           W1, b1, g1, be1, W2, b2, g2, be2,
           Wc1, bc1, gc1, bec1, Wc2, bc2):
    src = drug_adj[0]
    dst = drug_adj[1]
    dst3 = dst.reshape(_NW, _NWIN, _WIN)
    pk = jnp.arange(_WIN, dtype=jnp.int32)
    pad_src = jnp.broadcast_to(pk * 125, (_NW, _WIN))
    pad_dst = jnp.broadcast_to(_N + (pk % 16), (_NW, _WIN))
    srcp = jnp.concatenate([src.reshape(_NW, _EPT), pad_src], axis=1)
    dstp = jnp.concatenate([dst.reshape(_NW, _EPT), pad_dst], axis=1)
    idx_flat = jnp.stack([srcp.reshape(_NW, _NWIN2, _WIN),
                          dstp.reshape(_NW, _NWIN2, _WIN)],
                         axis=2).reshape(_NW * _NWIN2, 2, _WIN)

    degp = _deg_call(dst3).reshape(_NC, _N)
    dinv = lax.rsqrt(degp[0] + degp[1] + 1.0)[:, None]

    ones_r = jnp.ones((1, _D), jnp.float32)
    zeros_r = jnp.zeros((1, _D), jnp.float32)

    y1 = _mm_scaled(drug_feature, W1, dinv, ones_r, zeros_r)
    acc1 = _scatter_call(idx_flat, y1)
    r1, s1, q1 = _post_call(acc1, y1, dinv, b1.reshape(1, _D))
    m1 = s1 / _N
    v1 = q1 / _N - m1 * m1
    a1 = g1.reshape(1, _D) * lax.rsqrt(v1 + 1e-5)
    c1 = be1.reshape(1, _D) - m1 * a1

    y2 = _mm_scaled(r1, W2, dinv, a1, c1)
    acc2 = _scatter_call(idx_flat, y2)
    r2, s2, q2 = _post_call(acc2, y2, dinv, b2.reshape(1, _D))
    m2 = s2 / _N
    v2 = q2 / _N - m2 * m2
    a2 = (g2.reshape(1, _D) * lax.rsqrt(v2 + 1e-5)).reshape(_D)
    c2 = (be2.reshape(1, _D) - m2 * (g2.reshape(1, _D) * lax.rsqrt(v2 + 1e-5)
                                     )).reshape(_D)

    partials = _segmax_call(r2, ibatch, a2, c2)

    gexpr_p = jnp.pad(gexpr_data, ((0, 0), (0, 7)))
    Wc1_p = jnp.pad(Wc1, ((0, 7), (0, 0)))
    x_drug, xc = _final_call(
        partials, gexpr_p, Wc1_p, bc1.reshape(1, _D), gc1.reshape(1, _D),
        bec1.reshape(1, _D), Wc2, bc2.reshape(1, _D))
    return (x_drug, xc)


# speedup vs baseline: 31.7359x; 1.0297x over previous
import functools

import jax
import jax.numpy as jnp
from jax import lax
from jax.experimental import pallas as pl
from jax.experimental.pallas import tpu as pltpu
from jax.experimental.pallas import tpu_sc as plsc

_N = 10000
_E = 320000
_D = 128
_B = 128
_NC = 2
_NS = 16
_NW = _NC * _NS
_EPT = _E // _NW
_WIN = 80
_NWIN = _EPT // _WIN
_RPS = _N // _NS
_ZCH = 624
_SEG = 320
_R = 3
_Q = 6
_NWIN2 = 126
_NACC = _N + 16
_BM = 1000


def _sc_mesh():
    return plsc.VectorSubcoreMesh(
        core_axis_name="c", subcore_axis_name="s",
        num_cores=_NC, num_subcores=_NS)


def _deg_call(dst3):
    @functools.partial(
        pl.kernel,
        out_type=jax.ShapeDtypeStruct((_NC * _N,), jnp.float32),
        mesh=_sc_mesh(),
        scratch_types=[
            pltpu.VMEM((_NWIN, _WIN), jnp.int32),
            pltpu.VMEM((_WIN,), jnp.float32),
            pltpu.VMEM((_ZCH,), jnp.float32),
            pltpu.VMEM_SHARED((_N,), jnp.float32),
            pltpu.SemaphoreType.DMA,
        ],
    )
    def deg_kernel(dst_hbm, out_hbm, idx_v, ones_v, z_v, deg_sh, sem):
        c = lax.axis_index("c")
        s = lax.axis_index("s")
        w = s * _NC + c

        def fill_ones(i, _):
            ones_v[pl.ds(i * 16, 16)] = jnp.full((16,), 1.0, jnp.float32)
            return 0
        lax.fori_loop(0, _WIN // 16, fill_ones, 0)

        def fill_zeros(i, _):
            z_v[pl.ds(i * 16, 16)] = jnp.zeros((16,), jnp.float32)
            return 0
        lax.fori_loop(0, _ZCH // 16, fill_zeros, 0)

        pltpu.sync_copy(z_v, deg_sh.at[pl.ds(s * _ZCH, _ZCH)])

        @pl.when(s == 0)
        def _():
            pltpu.sync_copy(z_v.at[pl.ds(0, 16)],
                            deg_sh.at[pl.ds(_NS * _ZCH, _N - _NS * _ZCH)])
        plsc.subcore_barrier()

        pltpu.sync_copy(dst_hbm.at[w], idx_v)

        def win(i, _):
            pltpu.async_copy(ones_v, deg_sh.at[idx_v.at[i]], sem, add=True)
            return 0
        lax.fori_loop(0, _NWIN, win, 0)

        def drain(i, _):
            pltpu.make_async_copy(ones_v, deg_sh.at[idx_v.at[0]], sem).wait()
            return 0
        lax.fori_loop(0, _NWIN, drain, 0)
        plsc.subcore_barrier()

        pltpu.sync_copy(deg_sh.at[pl.ds(s * _ZCH, _ZCH)], z_v)
        pltpu.sync_copy(z_v, out_hbm.at[pl.ds(c * _N + s * _ZCH, _ZCH)])

        @pl.when(s == 0)
        def _():
            tail = _N - _NS * _ZCH
            pltpu.sync_copy(deg_sh.at[pl.ds(_NS * _ZCH, tail)],
                            z_v.at[pl.ds(0, tail)])
            pltpu.sync_copy(z_v.at[pl.ds(0, tail)],
                            out_hbm.at[pl.ds(c * _N + _NS * _ZCH, tail)])

    return deg_kernel(dst3)


def _scatter_call(idx_flat, y):
    @functools.partial(
        pl.kernel,
        out_type=jax.ShapeDtypeStruct((_NC, _N, _D), jnp.float32),
        mesh=_sc_mesh(),
        scratch_types=[
            pltpu.VMEM((_Q, 2, _WIN), jnp.int32),
        ] + [pltpu.VMEM((_WIN, _D), jnp.float32) for _ in range(_R)]
          + [pltpu.VMEM_SHARED((_NACC, _D), jnp.float32)]
          + [pltpu.SemaphoreType.DMA] * (2 * _R + _Q),
    )
    def scat_kernel(idx_hbm, y_hbm, out_hbm, idxb, *rest):
        rows = rest[:_R]
        acc_sh = rest[_R]
        sems = rest[_R + 1:]
        gsem = sems[:_R]
        ssem = sems[_R:2 * _R]
        isem = sems[2 * _R:]
        z_v = rows[0]
        c = lax.axis_index("c")
        s = lax.axis_index("s")
        w = s * _NC + c
        tb = w * _NWIN2
        base = s * _ZCH
        chunks = [(base + k * _WIN, _WIN) for k in range(_ZCH // _WIN)]
        rem = _ZCH - (_ZCH // _WIN) * _WIN
        if rem:
            chunks.append((base + (_ZCH // _WIN) * _WIN, rem))

        def zrow(i, _):
            for j in range(_D // 16):
                z_v[i, pl.ds(j * 16, 16)] = jnp.zeros((16,), jnp.float32)
            return 0
        lax.fori_loop(0, _WIN, zrow, 0)
        for off, ln in chunks:
            pltpu.sync_copy(z_v.at[pl.ds(0, ln)], acc_sh.at[pl.ds(off, ln)])

        tail_off = _NS * _ZCH
        tail = _N - tail_off

        @pl.when(s == 0)
        def _():
            pltpu.sync_copy(z_v.at[pl.ds(0, tail)],
                            acc_sh.at[pl.ds(tail_off, tail)])
        plsc.subcore_barrier()

        def idx_load(i, q):
            pltpu.async_copy(idx_hbm.at[tb + i], idxb.at[q], isem[q])

        def idx_wait(q):
            pltpu.make_async_copy(idx_hbm.at[tb], idxb.at[q],
                                  isem[q]).wait()

        def gather(i, j, q):
            pltpu.async_copy(y_hbm.at[idxb.at[q, 0]], rows[j], gsem[j])

        def gather_wait(j, q):
            pltpu.make_async_copy(y_hbm.at[idxb.at[q, 0]], rows[j],
                                  gsem[j]).wait()

        def scatter(j, q):
            pltpu.async_copy(rows[j], acc_sh.at[idxb.at[q, 1]], ssem[j],
                             add=True)

        def scatter_wait(j):
            pltpu.make_async_copy(rows[j], acc_sh.at[idxb.at[0, 1]],
                                  ssem[j]).wait()

        for i in range(4):
            idx_load(i, i)
        for i in range(2):
            idx_wait(i)
            gather(i, i, i)

        def win6(k, _):
            for jj in range(6):
                i = k * 6 + jj
                j = jj % _R
                q = jj % _Q
                jn = (jj + 2) % _R

                @pl.when(i >= 1)
                def _(jn=jn):
                    scatter_wait(jn)
                gather_wait(j, q)
                scatter(j, q)

                @pl.when(i + 4 < _NWIN2)
                def _(i=i, qn=(jj + 4) % _Q):
                    idx_load(i + 4, qn)

                @pl.when(i + 2 < _NWIN2)
                def _(i=i, jn=jn, qn2=(jj + 2) % _Q):
                    idx_wait(qn2)
                    gather(i + 2, jn, qn2)
            return 0
        lax.fori_loop(0, _NWIN2 // 6, win6, 0)
        scatter_wait((_NWIN2 - 1) % _R)
        plsc.subcore_barrier()

        for off, ln in chunks:
            pltpu.sync_copy(acc_sh.at[pl.ds(off, ln)], z_v.at[pl.ds(0, ln)])
            pltpu.sync_copy(z_v.at[pl.ds(0, ln)],
                            out_hbm.at[c, pl.ds(off, ln)])

        @pl.when(s == 0)
        def _():
            pltpu.sync_copy(acc_sh.at[pl.ds(tail_off, tail)],
                            z_v.at[pl.ds(0, tail)])
            pltpu.sync_copy(z_v.at[pl.ds(0, tail)],
                            out_hbm.at[c, pl.ds(tail_off, tail)])

    return scat_kernel(idx_flat, y)


def _segmax_call(r2, ibatch):
    @functools.partial(
        pl.kernel,
        out_type=jax.ShapeDtypeStruct((_NW, _B, _D), jnp.float32),
        mesh=_sc_mesh(),
        scratch_types=[
            pltpu.VMEM((_SEG, _D), jnp.float32),
            pltpu.VMEM((_SEG,), jnp.int32),
            pltpu.VMEM((_B, _D), jnp.float32),
            pltpu.SemaphoreType.DMA,
            pltpu.SemaphoreType.DMA,
        ],
    )
    def seg_kernel(r_hbm, ib_hbm, out_hbm,
                   rows_v, seg_v, out_v, sem, sem2):
        c = lax.axis_index("c")
        s = lax.axis_index("s")
        w = s * _NC + c
        start = jnp.minimum(w * _SEG, _N - _SEG)

        half = _SEG // 2
        h0 = pltpu.async_copy(r_hbm.at[pl.ds(start, half)],
                              rows_v.at[pl.ds(0, half)], sem)
        h1 = pltpu.async_copy(r_hbm.at[pl.ds(start + half, half)],
                              rows_v.at[pl.ds(half, half)], sem2)
        pltpu.sync_copy(ib_hbm.at[pl.ds(start, _SEG)], seg_v)

        neg = jnp.full((16,), -jnp.inf, jnp.float32)

        def initrow(i, _):
            for j in range(_D // 16):
                out_v[i, pl.ds(j * 16, 16)] = neg
            return 0
        lax.fori_loop(0, _B, initrow, 0)

        def row16(k, _):
            base16 = k * 16
            seg16 = seg_v[pl.ds(base16, 16)]
            for r in range(16):
                seg = seg16[r]
                for j in range(_D // 16):
                    sl = pl.ds(j * 16, 16)
                    val = rows_v[base16 + r, sl]
                    out_v[seg, sl] = jnp.maximum(out_v[seg, sl], val)
            return 0
        h0.wait()
        lax.fori_loop(0, half // 16, row16, 0)
        h1.wait()
        lax.fori_loop(half // 16, _SEG // 16, row16, 0)

        pltpu.sync_copy(out_v, out_hbm.at[w])

    return seg_kernel(r2, ibatch)


def _mm_scaled(x, W, dinv_col, a_row, c_row):
    def kern(x_ref, w_ref, d_ref, a_ref, c_ref, o_ref):
        xb = x_ref[...] * a_ref[...] + c_ref[...]
        o_ref[...] = d_ref[...] * jnp.dot(
            xb, w_ref[...], preferred_element_type=jnp.float32)

    return pl.pallas_call(
        kern,
        grid=(_N // _BM,),
        in_specs=[
            pl.BlockSpec((_BM, _D), lambda i: (i, 0)),
            pl.BlockSpec((_D, _D), lambda i: (0, 0)),
            pl.BlockSpec((_BM, 1), lambda i: (i, 0)),
            pl.BlockSpec((1, _D), lambda i: (0, 0)),
            pl.BlockSpec((1, _D), lambda i: (0, 0)),
        ],
        out_specs=pl.BlockSpec((_BM, _D), lambda i: (i, 0)),
        out_shape=jax.ShapeDtypeStruct((_N, _D), jnp.float32),
    )(x, W, dinv_col, a_row, c_row)


def _post_phases(acc_ref, y_ref, d_ref, b_ref, g_ref, be_ref, o_ref,
                 r_sc, s_sc, q_sc, emit):
    p = pl.program_id(0)
    i = pl.program_id(1)

    @pl.when(p == 0)
    def _():
        z = d_ref[...] * (acc_ref[0] + acc_ref[1] + y_ref[...]) + b_ref[...]
        r = jnp.maximum(z, 0.0)
        r_sc[i] = r

        @pl.when(i == 0)
        def _():
            s_sc[...] = jnp.zeros_like(s_sc)
            q_sc[...] = jnp.zeros_like(q_sc)

        s_sc[...] += jnp.sum(r, axis=0, keepdims=True)
        q_sc[...] += jnp.sum(r * r, axis=0, keepdims=True)

    @pl.when(p == 1)
    def _():
        m = s_sc[...] / _N
        v = q_sc[...] / _N - m * m
        a = g_ref[...] * lax.rsqrt(v + 1e-5)
        cc = be_ref[...] - m * a
        o_ref[...] = emit(r_sc[i] * a + cc)


def _post_specs():
    return dict(
        grid=(2, _N // _BM),
        in_specs=[
            pl.BlockSpec((_NC, _BM, _D), lambda p, i: (0, i * (1 - p), 0)),
            pl.BlockSpec((_BM, _D), lambda p, i: (i * (1 - p), 0)),
            pl.BlockSpec((_BM, 1), lambda p, i: (i, 0)),
            pl.BlockSpec((1, _D), lambda p, i: (0, 0)),
            pl.BlockSpec((1, _D), lambda p, i: (0, 0)),
            pl.BlockSpec((1, _D), lambda p, i: (0, 0)),
        ],
        out_specs=pl.BlockSpec((_BM, _D), lambda p, i: (p * i, 0)),
        out_shape=jax.ShapeDtypeStruct((_N, _D), jnp.float32),
        scratch_shapes=[
            pltpu.VMEM((_N // _BM, _BM, _D), jnp.float32),
            pltpu.VMEM((1, _D), jnp.float32),
            pltpu.VMEM((1, _D), jnp.float32),
        ],
    )


def _post_mm_call(accpair, y, dinv_col, b_row, g_row, be_row, W2):
    def kern(acc_ref, y_ref, d_ref, b_ref, g_ref, be_ref, w_ref, o_ref,
             r_sc, s_sc, q_sc):
        _post_phases(acc_ref, y_ref, d_ref, b_ref, g_ref, be_ref, o_ref,
                     r_sc, s_sc, q_sc,
                     lambda rn: d_ref[...] * jnp.dot(
                         rn, w_ref[...], preferred_element_type=jnp.float32))

    spec = _post_specs()
    spec["in_specs"] = spec["in_specs"] + [
        pl.BlockSpec((_D, _D), lambda p, i: (0, 0))]
    return pl.pallas_call(kern, **spec)(
        accpair, y, dinv_col, b_row, g_row, be_row, W2)


def _post_norm_call(accpair, y, dinv_col, b_row, g_row, be_row):
    def kern(acc_ref, y_ref, d_ref, b_ref, g_ref, be_ref, o_ref,
             r_sc, s_sc, q_sc):
        _post_phases(acc_ref, y_ref, d_ref, b_ref, g_ref, be_ref, o_ref,
                     r_sc, s_sc, q_sc, lambda rn: rn)

    return pl.pallas_call(kern, **_post_specs())(
        accpair, y, dinv_col, b_row, g_row, be_row)


def _final_call(partials, gexpr_p, Wc1_p, bc1r, gc1r, bec1r, Wc2, bc2r):
    dcp = gexpr_p.shape[1]

    def kern(p_ref, g_ref, w1_ref, b1_ref, g1_ref, be1_ref, w2_ref, b2_ref,
             xd_ref, xc_ref):
        xd_ref[...] = jnp.max(p_ref[...], axis=0)
        t = jnp.tanh(jnp.dot(g_ref[...], w1_ref[...],
                             preferred_element_type=jnp.float32) + b1_ref[...])
        m = jnp.mean(t, axis=0, keepdims=True)
        v = jnp.mean(t * t, axis=0, keepdims=True) - m * m
        tn = (t - m) * lax.rsqrt(v + 1e-5) * g1_ref[...] + be1_ref[...]
        xc_ref[...] = jnp.maximum(
            jnp.dot(tn, w2_ref[...], preferred_element_type=jnp.float32)
            + b2_ref[...], 0.0)

    return pl.pallas_call(
        kern,
        out_shape=[
            jax.ShapeDtypeStruct((_B, _D), jnp.float32),
            jax.ShapeDtypeStruct((_B, _D), jnp.float32),
        ],
    )(partials, gexpr_p, Wc1_p, bc1r, gc1r, bec1r, Wc2, bc2r)


def kernel(drug_feature, drug_adj, ibatch, gexpr_data,
           W1, b1, g1, be1, W2, b2, g2, be2,
           Wc1, bc1, gc1, bec1, Wc2, bc2):
    src = drug_adj[0]
    dst = drug_adj[1]
    dst3 = dst.reshape(_NW, _NWIN, _WIN)
    pk = jnp.arange(_WIN, dtype=jnp.int32)
    pad_src = jnp.broadcast_to(pk * 125, (_NW, _WIN))
    pad_dst = jnp.broadcast_to(_N + (pk % 16), (_NW, _WIN))
    srcp = jnp.concatenate([src.reshape(_NW, _EPT), pad_src], axis=1)
    dstp = jnp.concatenate([dst.reshape(_NW, _EPT), pad_dst], axis=1)
    idx_flat = jnp.stack([srcp.reshape(_NW, _NWIN2, _WIN),
                          dstp.reshape(_NW, _NWIN2, _WIN)],
                         axis=2).reshape(_NW * _NWIN2, 2, _WIN)

    degp = _deg_call(dst3).reshape(_NC, _N)
    dinv = lax.rsqrt(degp[0] + degp[1] + 1.0)[:, None]

    ones_r = jnp.ones((1, _D), jnp.float32)
    zeros_r = jnp.zeros((1, _D), jnp.float32)

    y1 = _mm_scaled(drug_feature, W1, dinv, ones_r, zeros_r)
    acc1 = _scatter_call(idx_flat, y1)
    y2 = _post_mm_call(acc1, y1, dinv, b1.reshape(1, _D), g1.reshape(1, _D),
                       be1.reshape(1, _D), W2)

    acc2 = _scatter_call(idx_flat, y2)
    r2n = _post_norm_call(acc2, y2, dinv, b2.reshape(1, _D),
                          g2.reshape(1, _D), be2.reshape(1, _D))

    partials = _segmax_call(r2n, ibatch)

    gexpr_p = jnp.pad(gexpr_data, ((0, 0), (0, 7)))
    Wc1_p = jnp.pad(Wc1, ((0, 7), (0, 0)))
    x_drug, xc = _final_call(
        partials, gexpr_p, Wc1_p, bc1.reshape(1, _D), gc1.reshape(1, _D),
        bec1.reshape(1, _D), Wc2, bc2.reshape(1, _D))
    return (x_drug, xc)


# speedup vs baseline: 32.3805x; 1.0203x over previous
import functools

import jax
import jax.numpy as jnp
from jax import lax
from jax.experimental import pallas as pl
from jax.experimental.pallas import tpu as pltpu
from jax.experimental.pallas import tpu_sc as plsc

_N = 10000
_E = 320000
_D = 128
_B = 128
_NC = 2
_NS = 16
_NW = _NC * _NS
_EPT = _E // _NW
_WIN = 80
_NWIN = _EPT // _WIN
_RPS = _N // _NS
_ZCH = 624
_SEG = 320
_R = 3
_Q = 6
_NWIN2 = 126
_NACC = _N + 16
_BM = 1000


def _sc_mesh():
    return plsc.VectorSubcoreMesh(
        core_axis_name="c", subcore_axis_name="s",
        num_cores=_NC, num_subcores=_NS)


def _deg_call(dst3):
    @functools.partial(
        pl.kernel,
        out_type=jax.ShapeDtypeStruct((_NC * _N,), jnp.float32),
        mesh=_sc_mesh(),
        scratch_types=[
            pltpu.VMEM((_NWIN, _WIN), jnp.int32),
            pltpu.VMEM((_WIN,), jnp.float32),
            pltpu.VMEM((_ZCH,), jnp.float32),
            pltpu.VMEM_SHARED((_N,), jnp.float32),
            pltpu.SemaphoreType.DMA,
        ],
    )
    def deg_kernel(dst_hbm, out_hbm, idx_v, ones_v, z_v, deg_sh, sem):
        c = lax.axis_index("c")
        s = lax.axis_index("s")
        w = s * _NC + c

        def fill_ones(i, _):
            ones_v[pl.ds(i * 16, 16)] = jnp.full((16,), 1.0, jnp.float32)
            return 0
        lax.fori_loop(0, _WIN // 16, fill_ones, 0)

        def fill_zeros(i, _):
            z_v[pl.ds(i * 16, 16)] = jnp.zeros((16,), jnp.float32)
            return 0
        lax.fori_loop(0, _ZCH // 16, fill_zeros, 0)

        pltpu.sync_copy(z_v, deg_sh.at[pl.ds(s * _ZCH, _ZCH)])

        @pl.when(s == 0)
        def _():
            pltpu.sync_copy(z_v.at[pl.ds(0, 16)],
                            deg_sh.at[pl.ds(_NS * _ZCH, _N - _NS * _ZCH)])
        plsc.subcore_barrier()

        pltpu.sync_copy(dst_hbm.at[w], idx_v)

        def win(i, _):
            pltpu.async_copy(ones_v, deg_sh.at[idx_v.at[i]], sem, add=True)
            return 0
        lax.fori_loop(0, _NWIN, win, 0)

        def drain(i, _):
            pltpu.make_async_copy(ones_v, deg_sh.at[idx_v.at[0]], sem).wait()
            return 0
        lax.fori_loop(0, _NWIN, drain, 0)
        plsc.subcore_barrier()

        pltpu.sync_copy(deg_sh.at[pl.ds(s * _ZCH, _ZCH)], z_v)
        pltpu.sync_copy(z_v, out_hbm.at[pl.ds(c * _N + s * _ZCH, _ZCH)])

        @pl.when(s == 0)
        def _():
            tail = _N - _NS * _ZCH
            pltpu.sync_copy(deg_sh.at[pl.ds(_NS * _ZCH, tail)],
                            z_v.at[pl.ds(0, tail)])
            pltpu.sync_copy(z_v.at[pl.ds(0, tail)],
                            out_hbm.at[pl.ds(c * _N + _NS * _ZCH, tail)])

    return deg_kernel(dst3)


def _scatter_call(idx_flat, y):
    @functools.partial(
        pl.kernel,
        out_type=jax.ShapeDtypeStruct((_NC, _N, _D), jnp.float32),
        mesh=_sc_mesh(),
        scratch_types=[
            pltpu.VMEM((_Q, 2, _WIN), jnp.int32),
        ] + [pltpu.VMEM((_WIN, _D), jnp.float32) for _ in range(_R)]
          + [pltpu.VMEM_SHARED((_NACC, _D), jnp.float32)]
          + [pltpu.SemaphoreType.DMA] * (2 * _R + _Q),
    )
    def scat_kernel(idx_hbm, y_hbm, out_hbm, idxb, *rest):
        rows = rest[:_R]
        acc_sh = rest[_R]
        sems = rest[_R + 1:]
        gsem = sems[:_R]
        ssem = sems[_R:2 * _R]
        isem = sems[2 * _R:]
        z_v = rows[0]
        c = lax.axis_index("c")
        s = lax.axis_index("s")
        w = s * _NC + c
        tb = w * _NWIN2
        base = s * _ZCH
        chunks = [(base + k * _WIN, _WIN) for k in range(_ZCH // _WIN)]
        rem = _ZCH - (_ZCH // _WIN) * _WIN
        if rem:
            chunks.append((base + (_ZCH // _WIN) * _WIN, rem))

        def zrow(i, _):
            for j in range(_D // 16):
                z_v[i, pl.ds(j * 16, 16)] = jnp.zeros((16,), jnp.float32)
            return 0
        lax.fori_loop(0, _WIN, zrow, 0)
        for off, ln in chunks:
            pltpu.sync_copy(z_v.at[pl.ds(0, ln)], acc_sh.at[pl.ds(off, ln)])

        tail_off = _NS * _ZCH
        tail = _N - tail_off

        @pl.when(s == 0)
        def _():
            pltpu.sync_copy(z_v.at[pl.ds(0, tail)],
                            acc_sh.at[pl.ds(tail_off, tail)])
        plsc.subcore_barrier()

        def idx_load(i, q):
            pltpu.async_copy(idx_hbm.at[tb + i], idxb.at[q], isem[q])

        def idx_wait(q):
            pltpu.make_async_copy(idx_hbm.at[tb], idxb.at[q],
                                  isem[q]).wait()

        def gather(i, j, q):
            pltpu.async_copy(y_hbm.at[idxb.at[q, 0]], rows[j], gsem[j])

        def gather_wait(j, q):
            pltpu.make_async_copy(y_hbm.at[idxb.at[q, 0]], rows[j],
                                  gsem[j]).wait()

        def scatter(j, q):
            pltpu.async_copy(rows[j], acc_sh.at[idxb.at[q, 1]], ssem[j],
                             add=True)

        def scatter_wait(j):
            pltpu.make_async_copy(rows[j], acc_sh.at[idxb.at[0, 1]],
                                  ssem[j]).wait()

        for i in range(4):
            idx_load(i, i)
        for i in range(2):
            idx_wait(i)
            gather(i, i, i)

        def win6(k, _):
            for jj in range(6):
                i = k * 6 + jj
                j = jj % _R
                q = jj % _Q
                jn = (jj + 2) % _R

                @pl.when(i >= 1)
                def _(jn=jn):
                    scatter_wait(jn)
                gather_wait(j, q)
                scatter(j, q)

                @pl.when(i + 4 < _NWIN2)
                def _(i=i, qn=(jj + 4) % _Q):
                    idx_load(i + 4, qn)

                @pl.when(i + 2 < _NWIN2)
                def _(i=i, jn=jn, qn2=(jj + 2) % _Q):
                    idx_wait(qn2)
                    gather(i + 2, jn, qn2)
            return 0
        lax.fori_loop(0, _NWIN2 // 6, win6, 0)
        scatter_wait((_NWIN2 - 1) % _R)
        plsc.subcore_barrier()

        for off, ln in chunks:
            pltpu.sync_copy(acc_sh.at[pl.ds(off, ln)], z_v.at[pl.ds(0, ln)])
            pltpu.sync_copy(z_v.at[pl.ds(0, ln)],
                            out_hbm.at[c, pl.ds(off, ln)])

        @pl.when(s == 0)
        def _():
            pltpu.sync_copy(acc_sh.at[pl.ds(tail_off, tail)],
                            z_v.at[pl.ds(0, tail)])
            pltpu.sync_copy(z_v.at[pl.ds(0, tail)],
                            out_hbm.at[c, pl.ds(tail_off, tail)])

    return scat_kernel(idx_flat, y)


def _segmax_call(r2, ibatch):
    @functools.partial(
        pl.kernel,
        out_type=jax.ShapeDtypeStruct((_NW, _B, _D), jnp.float32),
        mesh=_sc_mesh(),
        scratch_types=[
            pltpu.VMEM((_SEG, _D), jnp.float32),
            pltpu.VMEM((_SEG,), jnp.int32),
            pltpu.VMEM((_B, _D), jnp.float32),
            pltpu.SemaphoreType.DMA,
            pltpu.SemaphoreType.DMA,
        ],
    )
    def seg_kernel(r_hbm, ib_hbm, out_hbm,
                   rows_v, seg_v, out_v, sem, sem2):
        c = lax.axis_index("c")
        s = lax.axis_index("s")
        w = s * _NC + c
        start = jnp.minimum(w * _SEG, _N - _SEG)

        half = _SEG // 2
        h0 = pltpu.async_copy(r_hbm.at[pl.ds(start, half)],
                              rows_v.at[pl.ds(0, half)], sem)
        h1 = pltpu.async_copy(r_hbm.at[pl.ds(start + half, half)],
                              rows_v.at[pl.ds(half, half)], sem2)
        pltpu.sync_copy(ib_hbm.at[pl.ds(start, _SEG)], seg_v)

        neg = jnp.full((16,), -jnp.inf, jnp.float32)

        def initrow(i, _):
            for j in range(_D // 16):
                out_v[i, pl.ds(j * 16, 16)] = neg
            return 0
        lax.fori_loop(0, _B, initrow, 0)

        neg8 = tuple(jnp.full((16,), -jnp.inf, jnp.float32)
                     for _ in range(_D // 16))

        def flush(seg, m):
            for j in range(_D // 16):
                sl = pl.ds(j * 16, 16)
                out_v[seg, sl] = jnp.maximum(out_v[seg, sl], m[j])

        def row16(k, carry):
            prev = carry[0]
            m = list(carry[1:])
            base16 = k * 16
            seg16 = seg_v[pl.ds(base16, 16)]
            for r in range(16):
                seg = seg16[r]
                changed = seg != prev

                @pl.when(changed)
                def _(prev=prev, m=tuple(m)):
                    flush(prev, m)
                for j in range(_D // 16):
                    val = rows_v[base16 + r, pl.ds(j * 16, 16)]
                    m[j] = jnp.where(changed, val, jnp.maximum(m[j], val))
                prev = seg
            return (prev, *m)

        h0.wait()
        seg0 = seg_v[pl.ds(0, 16)][0]
        carry = lax.fori_loop(0, half // 16, row16, (seg0, *neg8))
        h1.wait()
        carry = lax.fori_loop(half // 16, _SEG // 16, row16, carry)
        flush(carry[0], carry[1:])

        pltpu.sync_copy(out_v, out_hbm.at[w])

    return seg_kernel(r2, ibatch)


def _mm_scaled(x, W, dinv_col, a_row, c_row):
    def kern(x_ref, w_ref, d_ref, a_ref, c_ref, o_ref):
        xb = x_ref[...] * a_ref[...] + c_ref[...]
        o_ref[...] = d_ref[...] * jnp.dot(
            xb, w_ref[...], preferred_element_type=jnp.float32)

    return pl.pallas_call(
        kern,
        grid=(_N // _BM,),
        in_specs=[
            pl.BlockSpec((_BM, _D), lambda i: (i, 0)),
            pl.BlockSpec((_D, _D), lambda i: (0, 0)),
            pl.BlockSpec((_BM, 1), lambda i: (i, 0)),
            pl.BlockSpec((1, _D), lambda i: (0, 0)),
            pl.BlockSpec((1, _D), lambda i: (0, 0)),
        ],
        out_specs=pl.BlockSpec((_BM, _D), lambda i: (i, 0)),
        out_shape=jax.ShapeDtypeStruct((_N, _D), jnp.float32),
    )(x, W, dinv_col, a_row, c_row)


def _post_phases(acc_ref, y_ref, d_ref, b_ref, g_ref, be_ref, o_ref,
                 r_sc, s_sc, q_sc, emit):
    p = pl.program_id(0)
    i = pl.program_id(1)

    @pl.when(p == 0)
    def _():
        z = d_ref[...] * (acc_ref[0] + acc_ref[1] + y_ref[...]) + b_ref[...]
        r = jnp.maximum(z, 0.0)
        r_sc[i] = r

        @pl.when(i == 0)
        def _():
            s_sc[...] = jnp.zeros_like(s_sc)
            q_sc[...] = jnp.zeros_like(q_sc)

        s_sc[...] += jnp.sum(r, axis=0, keepdims=True)
        q_sc[...] += jnp.sum(r * r, axis=0, keepdims=True)

    @pl.when(p == 1)
    def _():
        m = s_sc[...] / _N
        v = q_sc[...] / _N - m * m
        a = g_ref[...] * lax.rsqrt(v + 1e-5)
        cc = be_ref[...] - m * a
        o_ref[...] = emit(r_sc[i] * a + cc)


def _post_specs():
    return dict(
        grid=(2, _N // _BM),
        in_specs=[
            pl.BlockSpec((_NC, _BM, _D), lambda p, i: (0, i * (1 - p), 0)),
            pl.BlockSpec((_BM, _D), lambda p, i: (i * (1 - p), 0)),
            pl.BlockSpec((_BM, 1), lambda p, i: (i, 0)),
            pl.BlockSpec((1, _D), lambda p, i: (0, 0)),
            pl.BlockSpec((1, _D), lambda p, i: (0, 0)),
            pl.BlockSpec((1, _D), lambda p, i: (0, 0)),
        ],
        out_specs=pl.BlockSpec((_BM, _D), lambda p, i: (p * i, 0)),
        out_shape=jax.ShapeDtypeStruct((_N, _D), jnp.float32),
        scratch_shapes=[
            pltpu.VMEM((_N // _BM, _BM, _D), jnp.float32),
            pltpu.VMEM((1, _D), jnp.float32),
            pltpu.VMEM((1, _D), jnp.float32),
        ],
    )


def _post_mm_call(accpair, y, dinv_col, b_row, g_row, be_row, W2):
    def kern(acc_ref, y_ref, d_ref, b_ref, g_ref, be_ref, w_ref, o_ref,
             r_sc, s_sc, q_sc):
        _post_phases(acc_ref, y_ref, d_ref, b_ref, g_ref, be_ref, o_ref,
                     r_sc, s_sc, q_sc,
                     lambda rn: d_ref[...] * jnp.dot(
                         rn, w_ref[...], preferred_element_type=jnp.float32))

    spec = _post_specs()
    spec["in_specs"] = spec["in_specs"] + [
        pl.BlockSpec((_D, _D), lambda p, i: (0, 0))]
    return pl.pallas_call(kern, **spec)(
        accpair, y, dinv_col, b_row, g_row, be_row, W2)


def _post_norm_call(accpair, y, dinv_col, b_row, g_row, be_row):
    def kern(acc_ref, y_ref, d_ref, b_ref, g_ref, be_ref, o_ref,
             r_sc, s_sc, q_sc):
        _post_phases(acc_ref, y_ref, d_ref, b_ref, g_ref, be_ref, o_ref,
                     r_sc, s_sc, q_sc, lambda rn: rn)

    return pl.pallas_call(kern, **_post_specs())(
        accpair, y, dinv_col, b_row, g_row, be_row)


def _final_call(partials, gexpr_p, Wc1_p, bc1r, gc1r, bec1r, Wc2, bc2r):
    dcp = gexpr_p.shape[1]

    def kern(p_ref, g_ref, w1_ref, b1_ref, g1_ref, be1_ref, w2_ref, b2_ref,
             xd_ref, xc_ref):
        xd_ref[...] = jnp.max(p_ref[...], axis=0)
        t = jnp.tanh(jnp.dot(g_ref[...], w1_ref[...],
                             preferred_element_type=jnp.float32) + b1_ref[...])
        m = jnp.mean(t, axis=0, keepdims=True)
        v = jnp.mean(t * t, axis=0, keepdims=True) - m * m
        tn = (t - m) * lax.rsqrt(v + 1e-5) * g1_ref[...] + be1_ref[...]
        xc_ref[...] = jnp.maximum(
            jnp.dot(tn, w2_ref[...], preferred_element_type=jnp.float32)
            + b2_ref[...], 0.0)

    return pl.pallas_call(
        kern,
        out_shape=[
            jax.ShapeDtypeStruct((_B, _D), jnp.float32),
            jax.ShapeDtypeStruct((_B, _D), jnp.float32),
        ],
    )(partials, gexpr_p, Wc1_p, bc1r, gc1r, bec1r, Wc2, bc2r)


def kernel(drug_feature, drug_adj, ibatch, gexpr_data,
           W1, b1, g1, be1, W2, b2, g2, be2,
           Wc1, bc1, gc1, bec1, Wc2, bc2):
    src = drug_adj[0]
    dst = drug_adj[1]
    dst3 = dst.reshape(_NW, _NWIN, _WIN)
    pk = jnp.arange(_WIN, dtype=jnp.int32)
    pad_src = jnp.broadcast_to(pk * 125, (_NW, _WIN))
    pad_dst = jnp.broadcast_to(_N + (pk % 16), (_NW, _WIN))
    srcp = jnp.concatenate([src.reshape(_NW, _EPT), pad_src], axis=1)
    dstp = jnp.concatenate([dst.reshape(_NW, _EPT), pad_dst], axis=1)
    idx_flat = jnp.stack([srcp.reshape(_NW, _NWIN2, _WIN),
                          dstp.reshape(_NW, _NWIN2, _WIN)],
                         axis=2).reshape(_NW * _NWIN2, 2, _WIN)

    degp = _deg_call(dst3).reshape(_NC, _N)
    dinv = lax.rsqrt(degp[0] + degp[1] + 1.0)[:, None]

    ones_r = jnp.ones((1, _D), jnp.float32)
    zeros_r = jnp.zeros((1, _D), jnp.float32)

    y1 = _mm_scaled(drug_feature, W1, dinv, ones_r, zeros_r)
    acc1 = _scatter_call(idx_flat, y1)
    y2 = _post_mm_call(acc1, y1, dinv, b1.reshape(1, _D), g1.reshape(1, _D),
                       be1.reshape(1, _D), W2)

    acc2 = _scatter_call(idx_flat, y2)
    r2n = _post_norm_call(acc2, y2, dinv, b2.reshape(1, _D),
                          g2.reshape(1, _D), be2.reshape(1, _D))

    partials = _segmax_call(r2n, ibatch)

    gexpr_p = jnp.pad(gexpr_data, ((0, 0), (0, 7)))
    Wc1_p = jnp.pad(Wc1, ((0, 7), (0, 0)))
    x_drug, xc = _final_call(
        partials, gexpr_p, Wc1_p, bc1.reshape(1, _D), gc1.reshape(1, _D),
        bec1.reshape(1, _D), Wc2, bc2.reshape(1, _D))
    return (x_drug, xc)


# speedup vs baseline: 32.4685x; 1.0027x over previous
import functools

import jax
import jax.numpy as jnp
from jax import lax
from jax.experimental import pallas as pl
from jax.experimental.pallas import tpu as pltpu
from jax.experimental.pallas import tpu_sc as plsc

_N = 10000
_E = 320000
_D = 128
_B = 128
_NC = 2
_NS = 16
_NW = _NC * _NS
_EPT = _E // _NW
_WIN = 80
_NWIN = _EPT // _WIN
_RPS = _N // _NS
_ZCH = 624
_SEG = 320
_R = 3
_Q = 6
_NWIN2 = 126
_NACC = _N + 16
_BM = 1000


def _sc_mesh():
    return plsc.VectorSubcoreMesh(
        core_axis_name="c", subcore_axis_name="s",
        num_cores=_NC, num_subcores=_NS)


def _deg_call(dst3):
    @functools.partial(
        pl.kernel,
        out_type=jax.ShapeDtypeStruct((_NC * _N,), jnp.float32),
        mesh=_sc_mesh(),
        scratch_types=[
            pltpu.VMEM((_NWIN, _WIN), jnp.int32),
            pltpu.VMEM((_WIN,), jnp.float32),
            pltpu.VMEM((_ZCH,), jnp.float32),
            pltpu.VMEM_SHARED((_N,), jnp.float32),
            pltpu.SemaphoreType.DMA,
        ],
    )
    def deg_kernel(dst_hbm, out_hbm, idx_v, ones_v, z_v, deg_sh, sem):
        c = lax.axis_index("c")
        s = lax.axis_index("s")
        w = s * _NC + c

        def fill_ones(i, _):
            ones_v[pl.ds(i * 16, 16)] = jnp.full((16,), 1.0, jnp.float32)
            return 0
        lax.fori_loop(0, _WIN // 16, fill_ones, 0)

        def fill_zeros(i, _):
            z_v[pl.ds(i * 16, 16)] = jnp.zeros((16,), jnp.float32)
            return 0
        lax.fori_loop(0, _ZCH // 16, fill_zeros, 0)

        pltpu.sync_copy(z_v, deg_sh.at[pl.ds(s * _ZCH, _ZCH)])

        @pl.when(s == 0)
        def _():
            pltpu.sync_copy(z_v.at[pl.ds(0, 16)],
                            deg_sh.at[pl.ds(_NS * _ZCH, _N - _NS * _ZCH)])
        plsc.subcore_barrier()

        pltpu.sync_copy(dst_hbm.at[w], idx_v)

        def win(i, _):
            pltpu.async_copy(ones_v, deg_sh.at[idx_v.at[i]], sem, add=True)
            return 0
        lax.fori_loop(0, _NWIN, win, 0)

        def drain(i, _):
            pltpu.make_async_copy(ones_v, deg_sh.at[idx_v.at[0]], sem).wait()
            return 0
        lax.fori_loop(0, _NWIN, drain, 0)
        plsc.subcore_barrier()

        pltpu.sync_copy(deg_sh.at[pl.ds(s * _ZCH, _ZCH)], z_v)
        pltpu.sync_copy(z_v, out_hbm.at[pl.ds(c * _N + s * _ZCH, _ZCH)])

        @pl.when(s == 0)
        def _():
            tail = _N - _NS * _ZCH
            pltpu.sync_copy(deg_sh.at[pl.ds(_NS * _ZCH, tail)],
                            z_v.at[pl.ds(0, tail)])
            pltpu.sync_copy(z_v.at[pl.ds(0, tail)],
                            out_hbm.at[pl.ds(c * _N + _NS * _ZCH, tail)])

    return deg_kernel(dst3)


def _scatter_call(idx_flat, y):
    @functools.partial(
        pl.kernel,
        out_type=jax.ShapeDtypeStruct((_NC, _N, _D), jnp.float32),
        mesh=_sc_mesh(),
        scratch_types=[
            pltpu.VMEM((_Q, 2, _WIN), jnp.int32),
        ] + [pltpu.VMEM((_WIN, _D), jnp.float32) for _ in range(_R)]
          + [pltpu.VMEM_SHARED((_NACC, _D), jnp.float32)]
          + [pltpu.SemaphoreType.DMA] * (2 * _R + _Q),
    )
    def scat_kernel(idx_hbm, y_hbm, out_hbm, idxb, *rest):
        rows = rest[:_R]
        acc_sh = rest[_R]
        sems = rest[_R + 1:]
        gsem = sems[:_R]
        ssem = sems[_R:2 * _R]
        isem = sems[2 * _R:]
        z_v = rows[0]
        c = lax.axis_index("c")
        s = lax.axis_index("s")
        w = s * _NC + c
        tb = w * _NWIN2
        base = s * _ZCH
        chunks = [(base + k * _WIN, _WIN) for k in range(_ZCH // _WIN)]
        rem = _ZCH - (_ZCH // _WIN) * _WIN
        if rem:
            chunks.append((base + (_ZCH // _WIN) * _WIN, rem))

        def zrow(i, _):
            for j in range(_D // 16):
                z_v[i, pl.ds(j * 16, 16)] = jnp.zeros((16,), jnp.float32)
            return 0
        lax.fori_loop(0, _WIN, zrow, 0)
        for off, ln in chunks:
            pltpu.sync_copy(z_v.at[pl.ds(0, ln)], acc_sh.at[pl.ds(off, ln)])

        tail_off = _NS * _ZCH
        tail = _N - tail_off

        @pl.when(s == 0)
        def _():
            pltpu.sync_copy(z_v.at[pl.ds(0, tail)],
                            acc_sh.at[pl.ds(tail_off, tail)])

        def idx_load(i, q):
            pltpu.async_copy(idx_hbm.at[tb + i], idxb.at[q], isem[q])

        def idx_wait(q):
            pltpu.make_async_copy(idx_hbm.at[tb], idxb.at[q],
                                  isem[q]).wait()

        def gather(i, j, q):
            pltpu.async_copy(y_hbm.at[idxb.at[q, 0]], rows[j], gsem[j])

        def gather_wait(j, q):
            pltpu.make_async_copy(y_hbm.at[idxb.at[q, 0]], rows[j],
                                  gsem[j]).wait()

        def scatter(j, q):
            pltpu.async_copy(rows[j], acc_sh.at[idxb.at[q, 1]], ssem[j],
                             add=True)

        def scatter_wait(j):
            pltpu.make_async_copy(rows[j], acc_sh.at[idxb.at[0, 1]],
                                  ssem[j]).wait()

        for i in range(4):
            idx_load(i, i)
        for i in range(2):
            idx_wait(i)
            gather(i, i, i)
        plsc.subcore_barrier()

        def win6(k, _):
            for jj in range(6):
                i = k * 6 + jj
                j = jj % _R
                q = jj % _Q
                jn = (jj + 2) % _R

                @pl.when(i >= 1)
                def _(jn=jn):
                    scatter_wait(jn)
                gather_wait(j, q)
                scatter(j, q)

                @pl.when(i + 4 < _NWIN2)
                def _(i=i, qn=(jj + 4) % _Q):
                    idx_load(i + 4, qn)

                @pl.when(i + 2 < _NWIN2)
                def _(i=i, jn=jn, qn2=(jj + 2) % _Q):
                    idx_wait(qn2)
                    gather(i + 2, jn, qn2)
            return 0
        lax.fori_loop(0, _NWIN2 // 6, win6, 0)
        scatter_wait((_NWIN2 - 1) % _R)
        plsc.subcore_barrier()

        for off, ln in chunks:
            pltpu.sync_copy(acc_sh.at[pl.ds(off, ln)], z_v.at[pl.ds(0, ln)])
            pltpu.sync_copy(z_v.at[pl.ds(0, ln)],
                            out_hbm.at[c, pl.ds(off, ln)])

        @pl.when(s == 0)
        def _():
            pltpu.sync_copy(acc_sh.at[pl.ds(tail_off, tail)],
                            z_v.at[pl.ds(0, tail)])
            pltpu.sync_copy(z_v.at[pl.ds(0, tail)],
                            out_hbm.at[c, pl.ds(tail_off, tail)])

    return scat_kernel(idx_flat, y)


def _segmax_call(r2, ibatch):
    @functools.partial(
        pl.kernel,
        out_type=jax.ShapeDtypeStruct((_NW, _B, _D), jnp.float32),
        mesh=_sc_mesh(),
        scratch_types=[
            pltpu.VMEM((_SEG, _D), jnp.float32),
            pltpu.VMEM((_SEG,), jnp.int32),
            pltpu.VMEM((_B, _D), jnp.float32),
            pltpu.SemaphoreType.DMA,
            pltpu.SemaphoreType.DMA,
        ],
    )
    def seg_kernel(r_hbm, ib_hbm, out_hbm,
                   rows_v, seg_v, out_v, sem, sem2):
        c = lax.axis_index("c")
        s = lax.axis_index("s")
        w = s * _NC + c
        start = jnp.minimum(w * _SEG, _N - _SEG)

        half = _SEG // 2
        h0 = pltpu.async_copy(r_hbm.at[pl.ds(start, half)],
                              rows_v.at[pl.ds(0, half)], sem)
        h1 = pltpu.async_copy(r_hbm.at[pl.ds(start + half, half)],
                              rows_v.at[pl.ds(half, half)], sem2)
        pltpu.sync_copy(ib_hbm.at[pl.ds(start, _SEG)], seg_v)

        neg = jnp.full((16,), -jnp.inf, jnp.float32)

        def initrow(i, _):
            for j in range(_D // 16):
                out_v[i, pl.ds(j * 16, 16)] = neg
            return 0
        lax.fori_loop(0, _B, initrow, 0)

        neg8 = tuple(jnp.full((16,), -jnp.inf, jnp.float32)
                     for _ in range(_D // 16))

        def flush(seg, m):
            for j in range(_D // 16):
                sl = pl.ds(j * 16, 16)
                out_v[seg, sl] = jnp.maximum(out_v[seg, sl], m[j])

        def row16(k, carry):
            prev = carry[0]
            m = list(carry[1:])
            base16 = k * 16
            seg16 = seg_v[pl.ds(base16, 16)]
            for r in range(16):
                seg = seg16[r]
                changed = seg != prev

                @pl.when(changed)
                def _(prev=prev, m=tuple(m)):
                    flush(prev, m)
                for j in range(_D // 16):
                    val = rows_v[base16 + r, pl.ds(j * 16, 16)]
                    m[j] = jnp.where(changed, val, jnp.maximum(m[j], val))
                prev = seg
            return (prev, *m)

        h0.wait()
        seg0 = seg_v[pl.ds(0, 16)][0]
        carry = lax.fori_loop(0, half // 16, row16, (seg0, *neg8))
        h1.wait()
        carry = lax.fori_loop(half // 16, _SEG // 16, row16, carry)
        flush(carry[0], carry[1:])

        pltpu.sync_copy(out_v, out_hbm.at[w])

    return seg_kernel(r2, ibatch)


def _mm_scaled(x, W, dinv_col, a_row, c_row):
    def kern(x_ref, w_ref, d_ref, a_ref, c_ref, o_ref):
        xb = x_ref[...] * a_ref[...] + c_ref[...]
        o_ref[...] = d_ref[...] * jnp.dot(
            xb, w_ref[...], preferred_element_type=jnp.float32)

    return pl.pallas_call(
        kern,
        grid=(_N // _BM,),
        in_specs=[
            pl.BlockSpec((_BM, _D), lambda i: (i, 0)),
            pl.BlockSpec((_D, _D), lambda i: (0, 0)),
            pl.BlockSpec((_BM, 1), lambda i: (i, 0)),
            pl.BlockSpec((1, _D), lambda i: (0, 0)),
            pl.BlockSpec((1, _D), lambda i: (0, 0)),
        ],
        out_specs=pl.BlockSpec((_BM, _D), lambda i: (i, 0)),
        out_shape=jax.ShapeDtypeStruct((_N, _D), jnp.float32),
    )(x, W, dinv_col, a_row, c_row)


def _post_phases(acc_ref, y_ref, d_ref, b_ref, g_ref, be_ref, o_ref,
                 r_sc, s_sc, q_sc, emit):
    p = pl.program_id(0)
    i = pl.program_id(1)

    @pl.when(p == 0)
    def _():
        z = d_ref[...] * (acc_ref[0] + acc_ref[1] + y_ref[...]) + b_ref[...]
        r = jnp.maximum(z, 0.0)
        r_sc[i] = r

        @pl.when(i == 0)
        def _():
            s_sc[...] = jnp.zeros_like(s_sc)
            q_sc[...] = jnp.zeros_like(q_sc)

        s_sc[...] += jnp.sum(r, axis=0, keepdims=True)
        q_sc[...] += jnp.sum(r * r, axis=0, keepdims=True)

    @pl.when(p == 1)
    def _():
        m = s_sc[...] / _N
        v = q_sc[...] / _N - m * m
        a = g_ref[...] * lax.rsqrt(v + 1e-5)
        cc = be_ref[...] - m * a
        o_ref[...] = emit(r_sc[i] * a + cc)


def _post_specs():
    return dict(
        grid=(2, _N // _BM),
        in_specs=[
            pl.BlockSpec((_NC, _BM, _D), lambda p, i: (0, i * (1 - p), 0)),
            pl.BlockSpec((_BM, _D), lambda p, i: (i * (1 - p), 0)),
            pl.BlockSpec((_BM, 1), lambda p, i: (i, 0)),
            pl.BlockSpec((1, _D), lambda p, i: (0, 0)),
            pl.BlockSpec((1, _D), lambda p, i: (0, 0)),
            pl.BlockSpec((1, _D), lambda p, i: (0, 0)),
        ],
        out_specs=pl.BlockSpec((_BM, _D), lambda p, i: (p * i, 0)),
        out_shape=jax.ShapeDtypeStruct((_N, _D), jnp.float32),
        scratch_shapes=[
            pltpu.VMEM((_N // _BM, _BM, _D), jnp.float32),
            pltpu.VMEM((1, _D), jnp.float32),
            pltpu.VMEM((1, _D), jnp.float32),
        ],
    )


def _post_mm_call(accpair, y, dinv_col, b_row, g_row, be_row, W2):
    def kern(acc_ref, y_ref, d_ref, b_ref, g_ref, be_ref, w_ref, o_ref,
             r_sc, s_sc, q_sc):
        _post_phases(acc_ref, y_ref, d_ref, b_ref, g_ref, be_ref, o_ref,
                     r_sc, s_sc, q_sc,
                     lambda rn: d_ref[...] * jnp.dot(
                         rn, w_ref[...], preferred_element_type=jnp.float32))

    spec = _post_specs()
    spec["in_specs"] = spec["in_specs"] + [
        pl.BlockSpec((_D, _D), lambda p, i: (0, 0))]
    return pl.pallas_call(kern, **spec)(
        accpair, y, dinv_col, b_row, g_row, be_row, W2)


def _post_norm_call(accpair, y, dinv_col, b_row, g_row, be_row):
    def kern(acc_ref, y_ref, d_ref, b_ref, g_ref, be_ref, o_ref,
             r_sc, s_sc, q_sc):
        _post_phases(acc_ref, y_ref, d_ref, b_ref, g_ref, be_ref, o_ref,
                     r_sc, s_sc, q_sc, lambda rn: rn)

    return pl.pallas_call(kern, **_post_specs())(
        accpair, y, dinv_col, b_row, g_row, be_row)


def _cellline_call(gexpr_p, Wc1_p, bc1r, gc1r, bec1r, Wc2, bc2r):
    def kern(g_ref, w1_ref, b1_ref, g1_ref, be1_ref, w2_ref, b2_ref, xc_ref):
        t = jnp.tanh(jnp.dot(g_ref[...], w1_ref[...],
                             preferred_element_type=jnp.float32) + b1_ref[...])
        m = jnp.mean(t, axis=0, keepdims=True)
        v = jnp.mean(t * t, axis=0, keepdims=True) - m * m
        tn = (t - m) * lax.rsqrt(v + 1e-5) * g1_ref[...] + be1_ref[...]
        xc_ref[...] = jnp.maximum(
            jnp.dot(tn, w2_ref[...], preferred_element_type=jnp.float32)
            + b2_ref[...], 0.0)

    return pl.pallas_call(
        kern,
        out_shape=jax.ShapeDtypeStruct((_B, _D), jnp.float32),
    )(gexpr_p, Wc1_p, bc1r, gc1r, bec1r, Wc2, bc2r)


def _maxcomb_call(partials):
    def kern(p_ref, xd_ref):
        xd_ref[...] = jnp.max(p_ref[...], axis=0)

    return pl.pallas_call(
        kern,
        out_shape=jax.ShapeDtypeStruct((_B, _D), jnp.float32),
    )(partials)


def kernel(drug_feature, drug_adj, ibatch, gexpr_data,
           W1, b1, g1, be1, W2, b2, g2, be2,
           Wc1, bc1, gc1, bec1, Wc2, bc2):
    src = drug_adj[0]
    dst = drug_adj[1]
    dst3 = dst.reshape(_NW, _NWIN, _WIN)
    pk = jnp.arange(_WIN, dtype=jnp.int32)
    pad_src = jnp.broadcast_to(pk * 125, (_NW, _WIN))
    pad_dst = jnp.broadcast_to(_N + (pk % 16), (_NW, _WIN))
    srcp = jnp.concatenate([src.reshape(_NW, _EPT), pad_src], axis=1)
    dstp = jnp.concatenate([dst.reshape(_NW, _EPT), pad_dst], axis=1)
    idx_flat = jnp.stack([srcp.reshape(_NW, _NWIN2, _WIN),
                          dstp.reshape(_NW, _NWIN2, _WIN)],
                         axis=2).reshape(_NW * _NWIN2, 2, _WIN)

    degp = _deg_call(dst3).reshape(_NC, _N)
    dinv = lax.rsqrt(degp[0] + degp[1] + 1.0)[:, None]

    ones_r = jnp.ones((1, _D), jnp.float32)
    zeros_r = jnp.zeros((1, _D), jnp.float32)

    y1 = _mm_scaled(drug_feature, W1, dinv, ones_r, zeros_r)
    acc1 = _scatter_call(idx_flat, y1)
    y2 = _post_mm_call(acc1, y1, dinv, b1.reshape(1, _D), g1.reshape(1, _D),
                       be1.reshape(1, _D), W2)

    acc2 = _scatter_call(idx_flat, y2)
    r2n = _post_norm_call(acc2, y2, dinv, b2.reshape(1, _D),
                          g2.reshape(1, _D), be2.reshape(1, _D))

    partials = _segmax_call(r2n, ibatch)

    gexpr_p = jnp.pad(gexpr_data, ((0, 0), (0, 7)))
    Wc1_p = jnp.pad(Wc1, ((0, 7), (0, 0)))
    xc = _cellline_call(gexpr_p, Wc1_p, bc1.reshape(1, _D),
                        gc1.reshape(1, _D), bec1.reshape(1, _D), Wc2,
                        bc2.reshape(1, _D))
    x_drug = _maxcomb_call(partials)
    return (x_drug, xc)
